# Initial kernel scaffold; baseline (speedup 1.0000x reference)
import functools

import jax
import jax.numpy as jnp
from jax import lax
from jax.experimental import pallas as pl
from jax.experimental.pallas import tpu as pltpu
from jax.experimental.pallas import tpu_sc as plsc

N = 10000
E = 160000
D = 128
B = 2
TS = 4
NC = 2
NSUB = 16
K = 128
RPT = N // NSUB
IOB = 125

f32 = jnp.float32
i32 = jnp.int32

_sc_mesh = plsc.VectorSubcoreMesh(core_axis_name="c", subcore_axis_name="s")


def _zero_vmem_rows(buf, nrows):
    z = jnp.zeros((16,), f32)

    @pl.loop(0, nrows)
    def _(r):
        for i in range(D // 16):
            buf[r, pl.ds(i * 16, 16)] = z


def _sc_struct_body(table, src, dst, et, out, acc, si, di, ti, gi, rows,
                    si_t, di_t, ti_t, gi_t, rows_t, iobuf, gsem):
    c = lax.axis_index("c")
    s = lax.axis_index("s")
    b = c

    _zero_vmem_rows(iobuf, IOB)
    r0 = s * RPT
    for k in range(RPT // IOB):
        pltpu.sync_copy(iobuf, acc.at[pl.ds(r0 + k * IOB, IOB)])
    plsc.subcore_barrier()

    ept = E // NSUB
    base0 = s * ept
    nfull = ept // K
    tail = ept - nfull * K

    def chunk(base, kk, sib, dib, tib, gib, rb):
        pltpu.sync_copy(src.at[pl.ds(base, kk)], sib.at[0])
        pltpu.sync_copy(dst.at[pl.ds(base, kk)], dib.at[0])
        pltpu.sync_copy(et.at[pl.ds(base, kk)], tib.at[0])
        for i in range(kk // 16):
            sl = pl.ds(i * 16, 16)
            gib[0, sl] = (b * TS + tib[0, sl]) * N + sib[0, sl]
        pltpu.async_copy(table.at[gib.at[0]], rb, gsem).wait()
        pltpu.sync_copy(rb, acc.at[dib.at[0]], add=True)

    @pl.loop(0, nfull)
    def _(j):
        chunk(base0 + j * K, K, si, di, ti, gi, rows)

    if tail:
        chunk(base0 + nfull * K, tail, si_t, di_t, ti_t, gi_t, rows_t)

    plsc.subcore_barrier()
    for k in range(RPT // IOB):
        rr = r0 + k * IOB
        pltpu.sync_copy(acc.at[pl.ds(rr, IOB)], iobuf)
        pltpu.sync_copy(iobuf, out.at[pl.ds(b * N + rr, IOB)])


_sc_struct = pl.kernel(
    _sc_struct_body,
    out_type=jax.ShapeDtypeStruct((B * N, D), f32),
    mesh=_sc_mesh,
    scratch_types=[
        pltpu.VMEM_SHARED((N, D), f32),
        pltpu.VMEM((1, K), i32),
        pltpu.VMEM((1, K), i32),
        pltpu.VMEM((1, K), i32),
        pltpu.VMEM((1, K), i32),
        pltpu.VMEM((K, D), f32),
        pltpu.VMEM((1, 16), i32),
        pltpu.VMEM((1, 16), i32),
        pltpu.VMEM((1, 16), i32),
        pltpu.VMEM((1, 16), i32),
        pltpu.VMEM((16, D), f32),
        pltpu.VMEM((IOB, D), f32),
        pltpu.SemaphoreType.DMA,
    ],
    name="sc_struct_agg",
)


def _sc_text_body(table, src, dst, out, acc, si, di, rows,
                  si_t, di_t, rows_t, iobuf, gsem):
    c = lax.axis_index("c")
    s = lax.axis_index("s")
    wid = s * NC + c

    _zero_vmem_rows(iobuf, IOB)
    r0 = s * RPT
    for k in range(RPT // IOB):
        pltpu.sync_copy(iobuf, acc.at[pl.ds(r0 + k * IOB, IOB)])
    plsc.subcore_barrier()

    ept = E // (NC * NSUB)
    base0 = wid * ept
    nfull = ept // K
    tail = ept - nfull * K

    def chunk(base, kk, sib, dib, rb):
        pltpu.sync_copy(src.at[pl.ds(base, kk)], sib.at[0])
        pltpu.sync_copy(dst.at[pl.ds(base, kk)], dib.at[0])
        pltpu.async_copy(table.at[sib.at[0]], rb, gsem).wait()
        pltpu.sync_copy(rb, acc.at[dib.at[0]], add=True)

    @pl.loop(0, nfull)
    def _(j):
        chunk(base0 + j * K, K, si, di, rows)

    if tail:
        chunk(base0 + nfull * K, tail, si_t, di_t, rows_t)

    plsc.subcore_barrier()
    for k in range(RPT // IOB):
        rr = r0 + k * IOB
        pltpu.sync_copy(acc.at[pl.ds(rr, IOB)], iobuf)
        pltpu.sync_copy(iobuf, out.at[pl.ds(c * N + rr, IOB)])


_sc_text = pl.kernel(
    _sc_text_body,
    out_type=jax.ShapeDtypeStruct((NC * N, D), f32),
    mesh=_sc_mesh,
    scratch_types=[
        pltpu.VMEM_SHARED((N, D), f32),
        pltpu.VMEM((1, K), i32),
        pltpu.VMEM((1, K), i32),
        pltpu.VMEM((K, D), f32),
        pltpu.VMEM((1, 8), i32),
        pltpu.VMEM((1, 8), i32),
        pltpu.VMEM((8, D), f32),
        pltpu.VMEM((IOB, D), f32),
        pltpu.SemaphoreType.DMA,
    ],
    name="sc_text_agg",
)


BN = 1000


def _mlp_s0_body(q_ref, agg_ref, w_ref, b_ref, rel_ref, x1_ref, xr_ref):
    bb = pl.program_id(0)
    j = pl.program_id(1)
    w = w_ref[...]
    wb = w[D:]
    agg = agg_ref[0]
    y = jnp.dot(agg, wb, preferred_element_type=f32)
    csum = jnp.sum(w, axis=0)
    rid = j * BN + lax.broadcasted_iota(i32, (BN, 1), 0)
    y = y + jnp.where(rid == q_ref[bb], 1.0, 0.0) * csum[None, :]
    y = jnp.maximum(y + b_ref[...][None, :], 0.0)
    x1_ref[0] = y
    xr_ref[0] = rel_ref[...][:, None, :] * y[None]


def _mlp_s0(query, agg, w, bias, rel_next):
    return pl.pallas_call(
        _mlp_s0_body,
        grid=(B, N // BN),
        in_specs=[
            pl.BlockSpec(memory_space=pltpu.MemorySpace.SMEM),
            pl.BlockSpec((1, BN, D), lambda bb, j: (bb, j, 0)),
            pl.BlockSpec((2 * D, D), lambda bb, j: (0, 0)),
            pl.BlockSpec((D,), lambda bb, j: (0,)),
            pl.BlockSpec((TS, D), lambda bb, j: (0, 0)),
        ],
        out_specs=[
            pl.BlockSpec((1, BN, D), lambda bb, j: (bb, j, 0)),
            pl.BlockSpec((1, TS, BN, D), lambda bb, j: (bb, 0, j, 0)),
        ],
        out_shape=[
            jax.ShapeDtypeStruct((B, N, D), f32),
            jax.ShapeDtypeStruct((B, TS, N, D), f32),
        ],
    )(query, agg, w, bias, rel_next)


def _mlp_s1_body(q_ref, x_ref, agg_ref, w_ref, b_ref, h_ref):
    bb = pl.program_id(0)
    j = pl.program_id(1)
    w = w_ref[...]
    wt = w[:D]
    wb = w[D:]
    y = jnp.dot(x_ref[0], wt, preferred_element_type=f32)
    y = y + jnp.dot(agg_ref[0], wb, preferred_element_type=f32)
    csum = jnp.sum(wb, axis=0)
    rid = j * BN + lax.broadcasted_iota(i32, (BN, 1), 0)
    y = y + jnp.where(rid == q_ref[bb], 1.0, 0.0) * csum[None, :]
    h_ref[0] = jnp.maximum(y + b_ref[...][None, :], 0.0)


def _mlp_s1(query, x, agg, w, bias):
    return pl.pallas_call(
        _mlp_s1_body,
        grid=(B, N // BN),
        in_specs=[
            pl.BlockSpec(memory_space=pltpu.MemorySpace.SMEM),
            pl.BlockSpec((1, BN, D), lambda bb, j: (bb, j, 0)),
            pl.BlockSpec((1, BN, D), lambda bb, j: (bb, j, 0)),
            pl.BlockSpec((2 * D, D), lambda bb, j: (0, 0)),
            pl.BlockSpec((D,), lambda bb, j: (0,)),
        ],
        out_specs=pl.BlockSpec((1, BN, D), lambda bb, j: (bb, j, 0)),
        out_shape=jax.ShapeDtypeStruct((B, N, D), f32),
    )(query, x, agg, w, bias)


def _mlp_t_body(x_ref, part_ref, rti_ref, rel_ref, w_ref, b_ref, o_ref):
    w = w_ref[...]
    wt = w[:D]
    wb = w[D:]
    agg = (part_ref[0] + part_ref[1]) * rel_ref[...][0][None, :] \
        + rti_ref[...]
    y = jnp.dot(x_ref[...], wt, preferred_element_type=f32)
    y = y + jnp.dot(agg, wb, preferred_element_type=f32)
    o_ref[...] = jnp.maximum(y + b_ref[...][None, :], 0.0)


def _mlp_t(x, part, rti, rel, w, bias):
    return pl.pallas_call(
        _mlp_t_body,
        grid=(N // BN,),
        in_specs=[
            pl.BlockSpec((BN, D), lambda j: (j, 0)),
            pl.BlockSpec((NC, BN, D), lambda j: (0, j, 0)),
            pl.BlockSpec((BN, D), lambda j: (j, 0)),
            pl.BlockSpec((1, D), lambda j: (0, 0)),
            pl.BlockSpec((2 * D, D), lambda j: (0, 0)),
            pl.BlockSpec((D,), lambda j: (0,)),
        ],
        out_specs=pl.BlockSpec((BN, D), lambda j: (j, 0)),
        out_shape=jax.ShapeDtypeStruct((N, D), f32),
    )(x, part, rti, rel, w, bias)


def _fuse_body(h_ref, z_ref, w1_ref, b1_ref, w2_ref, b2_ref, o_ref):
    w1 = w1_ref[...]
    hid = jnp.dot(h_ref[0], w1[:D], preferred_element_type=f32)
    hid = hid + jnp.dot(z_ref[...], w1[D:], preferred_element_type=f32)
    hid = jnp.maximum(hid + b1_ref[...][None, :], 0.0)
    y = jnp.dot(hid, w2_ref[...], preferred_element_type=f32)
    o_ref[0] = y + b2_ref[...][None, :]


def _fuse(h, z, w1, b1, w2, b2):
    return pl.pallas_call(
        _fuse_body,
        grid=(B, N // BN),
        in_specs=[
            pl.BlockSpec((1, BN, D), lambda bb, j: (bb, j, 0)),
            pl.BlockSpec((BN, D), lambda bb, j: (j, 0)),
            pl.BlockSpec((2 * D, D), lambda bb, j: (0, 0)),
            pl.BlockSpec((D,), lambda bb, j: (0,)),
            pl.BlockSpec((D, D), lambda bb, j: (0, 0)),
            pl.BlockSpec((D,), lambda bb, j: (0,)),
        ],
        out_specs=pl.BlockSpec((1, BN, D), lambda bb, j: (bb, j, 0)),
        out_shape=jax.ShapeDtypeStruct((B, N, D), f32),
    )(h, z, w1, b1, w2, b2)


def kernel(query, edge_index, edge_type, text_edge_index, text_edge_type,
           rel_text_init, s_rel0, s_rel1, s_W0, s_b0, s_W1, s_b1,
           t_rel0, t_rel1, t_W0, t_b0, t_W1, t_b1,
           fuse_W1, fuse_b1, fuse_W2, fuse_b2):
    src_s = edge_index[0]
    dst_s = edge_index[1]
    src_t = text_edge_index[0]
    dst_t = text_edge_index[1]

    tab0 = jnp.zeros((B, TS, N, D), f32)
    tab0 = tab0.at[jnp.arange(B)[:, None], jnp.arange(TS)[None, :],
                   query[:, None], :].set(
        jnp.broadcast_to(s_rel0[None], (B, TS, D)))
    agg_s0 = _sc_struct(tab0.reshape(B * TS * N, D), src_s, dst_s, edge_type)
    x1, xrel1 = _mlp_s0(query, agg_s0.reshape(B, N, D), s_W0, s_b0, s_rel1)
    agg_s1 = _sc_struct(xrel1.reshape(B * TS * N, D), src_s, dst_s, edge_type)
    h = _mlp_s1(query, x1, agg_s1.reshape(B, N, D), s_W1, s_b1)

    part0 = _sc_text(rel_text_init, src_t, dst_t)
    x1t = _mlp_t(rel_text_init, part0.reshape(NC, N, D), rel_text_init,
                 t_rel0, t_W0, t_b0)
    part1 = _sc_text(x1t, src_t, dst_t)
    z = _mlp_t(x1t, part1.reshape(NC, N, D), rel_text_init,
               t_rel1, t_W1, t_b1)

    return _fuse(h, z, fuse_W1, fuse_b1, fuse_W2, fuse_b2)


# speedup vs baseline: 75.3730x; 75.3730x over previous
import functools

import jax
import jax.numpy as jnp
from jax import lax
from jax.experimental import pallas as pl
from jax.experimental.pallas import tpu as pltpu
from jax.experimental.pallas import tpu_sc as plsc

N = 10000
E = 160000
D = 128
B = 2
TS = 4
NC = 2
NSUB = 16
K = 128
WR_A = 632
WR_B = N - (NSUB - 1) * WR_A
CH = 104

f32 = jnp.float32
i32 = jnp.int32

@functools.cache
def _sc_mesh():
    return plsc.VectorSubcoreMesh(core_axis_name="c", subcore_axis_name="s",
                                  num_cores=NC, num_subcores=NSUB)


def _zero_vmem_rows(buf, nrows):
    z = jnp.zeros((16,), f32)

    @pl.loop(0, nrows)
    def _(r):
        for i in range(D // 16):
            buf[r, pl.ds(i * 16, 16)] = z


def _chunked(start, total, fn):
    nfull = total // CH
    tail = total - nfull * CH

    @pl.loop(0, nfull)
    def _(k):
        fn(start + k * CH, CH)

    if tail:
        fn(start + nfull * CH, tail)


def _per_tile_rows(s, fn):
    start = s * WR_A

    @pl.when(s < NSUB - 1)
    def _():
        fn(start, WR_A)

    @pl.when(s == NSUB - 1)
    def _():
        fn(start, WR_B)


def _zero_acc(acc, iobuf, s):
    _zero_vmem_rows(iobuf, CH)

    def z(off, sz):
        pltpu.sync_copy(iobuf.at[pl.ds(0, sz)], acc.at[pl.ds(off, sz)])

    _per_tile_rows(s, lambda start, total: _chunked(start, total, z))


def _acc_to_hbm(acc, iobuf, s, out, obase):
    def w(off, sz):
        pltpu.sync_copy(acc.at[pl.ds(off, sz)], iobuf.at[pl.ds(0, sz)])
        pltpu.sync_copy(iobuf.at[pl.ds(0, sz)],
                        out.at[pl.ds(obase + off, sz)])

    _per_tile_rows(s, lambda start, total: _chunked(start, total, w))


def _sc_struct_body(table, src, dst, et, out, acc, si, di, ti, gi, rows,
                    si_t, di_t, ti_t, gi_t, rows_t, iobuf, gsem):
    c = lax.axis_index("c")
    s = lax.axis_index("s")
    b = c

    _zero_acc(acc, iobuf, s)
    plsc.subcore_barrier()

    ept = E // NSUB
    base0 = s * ept
    nfull = ept // K
    tail = ept - nfull * K

    def chunk(base, kk, sib, dib, tib, gib, rb):
        pltpu.sync_copy(src.at[pl.ds(base, kk)], sib.at[0])
        pltpu.sync_copy(dst.at[pl.ds(base, kk)], dib.at[0])
        pltpu.sync_copy(et.at[pl.ds(base, kk)], tib.at[0])
        for i in range(kk // 16):
            sl = pl.ds(i * 16, 16)
            gib[0, sl] = (b * TS + tib[0, sl]) * N + sib[0, sl]
        pltpu.async_copy(table.at[gib.at[0]], rb, gsem).wait()
        pltpu.sync_copy(rb, acc.at[dib.at[0]], add=True)

    @pl.loop(0, nfull)
    def _(j):
        chunk(base0 + j * K, K, si, di, ti, gi, rows)

    if tail:
        chunk(base0 + nfull * K, tail, si_t, di_t, ti_t, gi_t, rows_t)

    plsc.subcore_barrier()
    _acc_to_hbm(acc, iobuf, s, out, b * N)


@functools.cache
def _sc_struct_kernel():
  return pl.kernel(
    _sc_struct_body,
    out_type=jax.ShapeDtypeStruct((B * N, D), f32),
    mesh=_sc_mesh(),
    scratch_types=[
        pltpu.VMEM_SHARED((N, D), f32),
        pltpu.VMEM((1, K), i32),
        pltpu.VMEM((1, K), i32),
        pltpu.VMEM((1, K), i32),
        pltpu.VMEM((1, K), i32),
        pltpu.VMEM((K, D), f32),
        pltpu.VMEM((1, 16), i32),
        pltpu.VMEM((1, 16), i32),
        pltpu.VMEM((1, 16), i32),
        pltpu.VMEM((1, 16), i32),
        pltpu.VMEM((16, D), f32),
        pltpu.VMEM((CH, D), f32),
        pltpu.SemaphoreType.DMA,
    ],
    name="sc_struct_agg",
  )


def _sc_struct(table, src, dst, et):
    return _sc_struct_kernel()(table, src, dst, et)


def _sc_text_body(table, src, dst, out, acc, si, di, rows,
                  si_t, di_t, rows_t, iobuf, gsem):
    c = lax.axis_index("c")
    s = lax.axis_index("s")
    wid = s * NC + c

    _zero_acc(acc, iobuf, s)
    plsc.subcore_barrier()

    ept = E // (NC * NSUB)
    base0 = wid * ept
    nfull = ept // K
    tail = ept - nfull * K

    def chunk(base, kk, sib, dib, rb):
        pltpu.sync_copy(src.at[pl.ds(base, kk)], sib.at[0])
        pltpu.sync_copy(dst.at[pl.ds(base, kk)], dib.at[0])
        pltpu.async_copy(table.at[sib.at[0]], rb, gsem).wait()
        pltpu.sync_copy(rb, acc.at[dib.at[0]], add=True)

    @pl.loop(0, nfull)
    def _(j):
        chunk(base0 + j * K, K, si, di, rows)

    if tail:
        chunk(base0 + nfull * K, tail, si_t, di_t, rows_t)

    plsc.subcore_barrier()
    _acc_to_hbm(acc, iobuf, s, out, c * N)


@functools.cache
def _sc_text_kernel():
  return pl.kernel(
    _sc_text_body,
    out_type=jax.ShapeDtypeStruct((NC * N, D), f32),
    mesh=_sc_mesh(),
    scratch_types=[
        pltpu.VMEM_SHARED((N, D), f32),
        pltpu.VMEM((1, K), i32),
        pltpu.VMEM((1, K), i32),
        pltpu.VMEM((K, D), f32),
        pltpu.VMEM((1, 8), i32),
        pltpu.VMEM((1, 8), i32),
        pltpu.VMEM((8, D), f32),
        pltpu.VMEM((CH, D), f32),
        pltpu.SemaphoreType.DMA,
    ],
    name="sc_text_agg",
  )


def _sc_text(table, src, dst):
    return _sc_text_kernel()(table, src, dst)


BN = 1000


def _mlp_s0_body(q_ref, agg_ref, w_ref, b_ref, rel_ref, x1_ref, xr_ref):
    bb = pl.program_id(0)
    j = pl.program_id(1)
    w = w_ref[...]
    wb = w[D:]
    agg = agg_ref[0]
    y = jnp.dot(agg, wb, preferred_element_type=f32)
    csum = jnp.sum(w, axis=0)
    rid = j * BN + lax.broadcasted_iota(i32, (BN, 1), 0)
    y = y + jnp.where(rid == q_ref[bb], 1.0, 0.0) * csum[None, :]
    y = jnp.maximum(y + b_ref[...][None, :], 0.0)
    x1_ref[0] = y
    xr_ref[0] = rel_ref[...][:, None, :] * y[None]


def _mlp_s0(query, agg, w, bias, rel_next):
    return pl.pallas_call(
        _mlp_s0_body,
        grid=(B, N // BN),
        in_specs=[
            pl.BlockSpec(memory_space=pltpu.MemorySpace.SMEM),
            pl.BlockSpec((1, BN, D), lambda bb, j: (bb, j, 0)),
            pl.BlockSpec((2 * D, D), lambda bb, j: (0, 0)),
            pl.BlockSpec((D,), lambda bb, j: (0,)),
            pl.BlockSpec((TS, D), lambda bb, j: (0, 0)),
        ],
        out_specs=[
            pl.BlockSpec((1, BN, D), lambda bb, j: (bb, j, 0)),
            pl.BlockSpec((1, TS, BN, D), lambda bb, j: (bb, 0, j, 0)),
        ],
        out_shape=[
            jax.ShapeDtypeStruct((B, N, D), f32),
            jax.ShapeDtypeStruct((B, TS, N, D), f32),
        ],
    )(query, agg, w, bias, rel_next)


def _mlp_s1_body(q_ref, x_ref, agg_ref, w_ref, b_ref, h_ref):
    bb = pl.program_id(0)
    j = pl.program_id(1)
    w = w_ref[...]
    wt = w[:D]
    wb = w[D:]
    y = jnp.dot(x_ref[0], wt, preferred_element_type=f32)
    y = y + jnp.dot(agg_ref[0], wb, preferred_element_type=f32)
    csum = jnp.sum(wb, axis=0)
    rid = j * BN + lax.broadcasted_iota(i32, (BN, 1), 0)
    y = y + jnp.where(rid == q_ref[bb], 1.0, 0.0) * csum[None, :]
    h_ref[0] = jnp.maximum(y + b_ref[...][None, :], 0.0)


def _mlp_s1(query, x, agg, w, bias):
    return pl.pallas_call(
        _mlp_s1_body,
        grid=(B, N // BN),
        in_specs=[
            pl.BlockSpec(memory_space=pltpu.MemorySpace.SMEM),
            pl.BlockSpec((1, BN, D), lambda bb, j: (bb, j, 0)),
            pl.BlockSpec((1, BN, D), lambda bb, j: (bb, j, 0)),
            pl.BlockSpec((2 * D, D), lambda bb, j: (0, 0)),
            pl.BlockSpec((D,), lambda bb, j: (0,)),
        ],
        out_specs=pl.BlockSpec((1, BN, D), lambda bb, j: (bb, j, 0)),
        out_shape=jax.ShapeDtypeStruct((B, N, D), f32),
    )(query, x, agg, w, bias)


def _mlp_t_body(x_ref, part_ref, rti_ref, rel_ref, w_ref, b_ref, o_ref):
    w = w_ref[...]
    wt = w[:D]
    wb = w[D:]
    agg = (part_ref[0] + part_ref[1]) * rel_ref[...][0][None, :] \
        + rti_ref[...]
    y = jnp.dot(x_ref[...], wt, preferred_element_type=f32)
    y = y + jnp.dot(agg, wb, preferred_element_type=f32)
    o_ref[...] = jnp.maximum(y + b_ref[...][None, :], 0.0)


def _mlp_t(x, part, rti, rel, w, bias):
    return pl.pallas_call(
        _mlp_t_body,
        grid=(N // BN,),
        in_specs=[
            pl.BlockSpec((BN, D), lambda j: (j, 0)),
            pl.BlockSpec((NC, BN, D), lambda j: (0, j, 0)),
            pl.BlockSpec((BN, D), lambda j: (j, 0)),
            pl.BlockSpec((1, D), lambda j: (0, 0)),
            pl.BlockSpec((2 * D, D), lambda j: (0, 0)),
            pl.BlockSpec((D,), lambda j: (0,)),
        ],
        out_specs=pl.BlockSpec((BN, D), lambda j: (j, 0)),
        out_shape=jax.ShapeDtypeStruct((N, D), f32),
    )(x, part, rti, rel, w, bias)


def _fuse_body(h_ref, z_ref, w1_ref, b1_ref, w2_ref, b2_ref, o_ref):
    w1 = w1_ref[...]
    hid = jnp.dot(h_ref[0], w1[:D], preferred_element_type=f32)
    hid = hid + jnp.dot(z_ref[...], w1[D:], preferred_element_type=f32)
    hid = jnp.maximum(hid + b1_ref[...][None, :], 0.0)
    y = jnp.dot(hid, w2_ref[...], preferred_element_type=f32)
    o_ref[0] = y + b2_ref[...][None, :]


def _fuse(h, z, w1, b1, w2, b2):
    return pl.pallas_call(
        _fuse_body,
        grid=(B, N // BN),
        in_specs=[
            pl.BlockSpec((1, BN, D), lambda bb, j: (bb, j, 0)),
            pl.BlockSpec((BN, D), lambda bb, j: (j, 0)),
            pl.BlockSpec((2 * D, D), lambda bb, j: (0, 0)),
            pl.BlockSpec((D,), lambda bb, j: (0,)),
            pl.BlockSpec((D, D), lambda bb, j: (0, 0)),
            pl.BlockSpec((D,), lambda bb, j: (0,)),
        ],
        out_specs=pl.BlockSpec((1, BN, D), lambda bb, j: (bb, j, 0)),
        out_shape=jax.ShapeDtypeStruct((B, N, D), f32),
    )(h, z, w1, b1, w2, b2)


def kernel(query, edge_index, edge_type, text_edge_index, text_edge_type,
           rel_text_init, s_rel0, s_rel1, s_W0, s_b0, s_W1, s_b1,
           t_rel0, t_rel1, t_W0, t_b0, t_W1, t_b1,
           fuse_W1, fuse_b1, fuse_W2, fuse_b2):
    src_s = edge_index[0]
    dst_s = edge_index[1]
    src_t = text_edge_index[0]
    dst_t = text_edge_index[1]

    tab0 = jnp.zeros((B, TS, N, D), f32)
    tab0 = tab0.at[jnp.arange(B)[:, None], jnp.arange(TS)[None, :],
                   query[:, None], :].set(
        jnp.broadcast_to(s_rel0[None], (B, TS, D)))
    agg_s0 = _sc_struct(tab0.reshape(B * TS * N, D), src_s, dst_s, edge_type)
    x1, xrel1 = _mlp_s0(query, agg_s0.reshape(B, N, D), s_W0, s_b0, s_rel1)
    agg_s1 = _sc_struct(xrel1.reshape(B * TS * N, D), src_s, dst_s, edge_type)
    h = _mlp_s1(query, x1, agg_s1.reshape(B, N, D), s_W1, s_b1)

    part0 = _sc_text(rel_text_init, src_t, dst_t)
    x1t = _mlp_t(rel_text_init, part0.reshape(NC, N, D), rel_text_init,
                 t_rel0, t_W0, t_b0)
    part1 = _sc_text(x1t, src_t, dst_t)
    z = _mlp_t(x1t, part1.reshape(NC, N, D), rel_text_init,
               t_rel1, t_W1, t_b1)

    return _fuse(h, z, fuse_W1, fuse_b1, fuse_W2, fuse_b2)


# speedup vs baseline: 155.1611x; 2.0586x over previous
import functools

import jax
import jax.numpy as jnp
from jax import lax
from jax.experimental import pallas as pl
from jax.experimental.pallas import tpu as pltpu
from jax.experimental.pallas import tpu_sc as plsc

N = 10000
E = 160000
D = 128
B = 2
TS = 4
NC = 2
NSUB = 16
K = 64
NBUF = 4
WR_A = 632
WR_B = N - (NSUB - 1) * WR_A
CH = 40

f32 = jnp.float32
i32 = jnp.int32

@functools.cache
def _sc_mesh():
    return plsc.VectorSubcoreMesh(core_axis_name="c", subcore_axis_name="s",
                                  num_cores=NC, num_subcores=NSUB)


def _zero_vmem_rows(buf, nrows):
    z = jnp.zeros((16,), f32)

    @pl.loop(0, nrows)
    def _(r):
        for i in range(D // 16):
            buf[r, pl.ds(i * 16, 16)] = z


def _chunked(start, total, fn):
    nfull = total // CH
    tail = total - nfull * CH

    @pl.loop(0, nfull)
    def _(k):
        fn(start + k * CH, CH)

    if tail:
        fn(start + nfull * CH, tail)


def _per_tile_rows(s, fn):
    start = s * WR_A

    @pl.when(s < NSUB - 1)
    def _():
        fn(start, WR_A)

    @pl.when(s == NSUB - 1)
    def _():
        fn(start, WR_B)


def _zero_acc(acc, iobuf, s):
    _zero_vmem_rows(iobuf, CH)

    def z(off, sz):
        pltpu.sync_copy(iobuf.at[pl.ds(0, sz)], acc.at[pl.ds(off, sz)])

    _per_tile_rows(s, lambda start, total: _chunked(start, total, z))


def _acc_to_hbm(acc, iobuf, s, out, obase):
    def w(off, sz):
        pltpu.sync_copy(acc.at[pl.ds(off, sz)], iobuf.at[pl.ds(0, sz)])
        pltpu.sync_copy(iobuf.at[pl.ds(0, sz)],
                        out.at[pl.ds(obase + off, sz)])

    _per_tile_rows(s, lambda start, total: _chunked(start, total, w))


def _edge_pipeline(table, src, dst, et, acc, si, di, ti, gi, rows,
                   isem, gsem, ssem, base0, nfull, gidx):
    gref = gi if gidx is not None else si

    def slot_of(j):
        return lax.rem(j + 2 * NBUF, NBUF)

    def idx_issue(j):
        slot = slot_of(j)
        base = base0 + j * K
        pltpu.async_copy(src.at[pl.ds(base, K)], si.at[slot], isem.at[slot])
        pltpu.async_copy(dst.at[pl.ds(base, K)], di.at[slot], isem.at[slot])
        if et is not None:
            pltpu.async_copy(et.at[pl.ds(base, K)], ti.at[slot],
                             isem.at[slot])

    def idx_wait(j):
        slot = slot_of(j)
        base = base0 + j * K
        pltpu.make_async_copy(src.at[pl.ds(base, K)], si.at[slot],
                              isem.at[slot]).wait()
        pltpu.make_async_copy(dst.at[pl.ds(base, K)], di.at[slot],
                              isem.at[slot]).wait()
        if et is not None:
            pltpu.make_async_copy(et.at[pl.ds(base, K)], ti.at[slot],
                                  isem.at[slot]).wait()

    def gather_issue(j):
        slot = slot_of(j)
        pltpu.async_copy(table.at[gref.at[slot]], rows.at[slot],
                         gsem.at[slot])

    def gather_wait(j):
        slot = slot_of(j)
        pltpu.make_async_copy(table.at[gref.at[slot]], rows.at[slot],
                              gsem.at[slot]).wait()

    def scatter_issue(j):
        slot = slot_of(j)
        pltpu.async_copy(rows.at[slot], acc.at[di.at[slot]], ssem.at[slot],
                         add=True)

    def scatter_wait(j):
        slot = slot_of(j)
        pltpu.make_async_copy(rows.at[slot], acc.at[di.at[slot]],
                              ssem.at[slot]).wait()

    def stage_front(j):
        idx_wait(j)
        if gidx is not None:
            gidx(slot_of(j))
        gather_issue(j)

    idx_issue(0)
    idx_issue(1)
    stage_front(0)

    @pl.loop(0, nfull)
    def _(j):
        @pl.when(j >= 2)
        def _():
            scatter_wait(j - 2)

        @pl.when(j + 2 < nfull)
        def _():
            idx_issue(j + 2)

        @pl.when(j + 1 < nfull)
        def _():
            stage_front(j + 1)

        gather_wait(j)
        scatter_issue(j)

    scatter_wait(nfull - 2)
    scatter_wait(nfull - 1)


def _sc_struct_body(table, src, dst, et, out, acc, si, di, ti, gi, rows,
                    si_t, di_t, ti_t, gi_t, rows_t, iobuf, isem, gsem, ssem):
    c = lax.axis_index("c")
    s = lax.axis_index("s")
    b = c

    _zero_acc(acc, iobuf, s)
    plsc.subcore_barrier()

    ept = E // NSUB
    base0 = s * ept
    nfull = ept // K
    tail = ept - nfull * K

    def gidx(slot):
        for i in range(K // 16):
            sl = pl.ds(i * 16, 16)
            gi[slot, sl] = (b * TS + ti[slot, sl]) * N + si[slot, sl]

    _edge_pipeline(table, src, dst, et, acc, si, di, ti, gi, rows,
                   isem, gsem, ssem, base0, nfull, gidx)

    if tail:
        base = base0 + nfull * K
        pltpu.sync_copy(src.at[pl.ds(base, tail)], si_t.at[0])
        pltpu.sync_copy(dst.at[pl.ds(base, tail)], di_t.at[0])
        pltpu.sync_copy(et.at[pl.ds(base, tail)], ti_t.at[0])
        for i in range(tail // 16):
            sl = pl.ds(i * 16, 16)
            gi_t[0, sl] = (b * TS + ti_t[0, sl]) * N + si_t[0, sl]
        pltpu.async_copy(table.at[gi_t.at[0]], rows_t, gsem.at[0]).wait()
        pltpu.sync_copy(rows_t, acc.at[di_t.at[0]], add=True)

    plsc.subcore_barrier()
    _acc_to_hbm(acc, iobuf, s, out, b * N)


@functools.cache
def _sc_struct_kernel():
  return pl.kernel(
    _sc_struct_body,
    out_type=jax.ShapeDtypeStruct((B * N, D), f32),
    mesh=_sc_mesh(),
    scratch_types=[
        pltpu.VMEM_SHARED((N, D), f32),
        pltpu.VMEM((NBUF, K), i32),
        pltpu.VMEM((NBUF, K), i32),
        pltpu.VMEM((NBUF, K), i32),
        pltpu.VMEM((NBUF, K), i32),
        pltpu.VMEM((NBUF, K, D), f32),
        pltpu.VMEM((1, 16), i32),
        pltpu.VMEM((1, 16), i32),
        pltpu.VMEM((1, 16), i32),
        pltpu.VMEM((1, 16), i32),
        pltpu.VMEM((16, D), f32),
        pltpu.VMEM((CH, D), f32),
        pltpu.SemaphoreType.DMA((NBUF,)),
        pltpu.SemaphoreType.DMA((NBUF,)),
        pltpu.SemaphoreType.DMA((NBUF,)),
    ],
    name="sc_struct_agg",
  )


def _sc_struct(table, src, dst, et):
    return _sc_struct_kernel()(table, src, dst, et)


def _sc_text_body(table, src, dst, out, acc, si, di, rows,
                  si_t, di_t, rows_t, iobuf, isem, gsem, ssem):
    c = lax.axis_index("c")
    s = lax.axis_index("s")
    wid = s * NC + c

    _zero_acc(acc, iobuf, s)
    plsc.subcore_barrier()

    ept = E // (NC * NSUB)
    base0 = wid * ept
    nfull = ept // K
    tail = ept - nfull * K

    _edge_pipeline(table, src, dst, None, acc, si, di, None, None, rows,
                   isem, gsem, ssem, base0, nfull, None)

    if tail:
        base = base0 + nfull * K
        pltpu.sync_copy(src.at[pl.ds(base, tail)], si_t.at[0])
        pltpu.sync_copy(dst.at[pl.ds(base, tail)], di_t.at[0])
        pltpu.async_copy(table.at[si_t.at[0]], rows_t, gsem.at[0]).wait()
        pltpu.sync_copy(rows_t, acc.at[di_t.at[0]], add=True)

    plsc.subcore_barrier()
    _acc_to_hbm(acc, iobuf, s, out, c * N)


@functools.cache
def _sc_text_kernel():
  return pl.kernel(
    _sc_text_body,
    out_type=jax.ShapeDtypeStruct((NC * N, D), f32),
    mesh=_sc_mesh(),
    scratch_types=[
        pltpu.VMEM_SHARED((N, D), f32),
        pltpu.VMEM((NBUF, K), i32),
        pltpu.VMEM((NBUF, K), i32),
        pltpu.VMEM((NBUF, K, D), f32),
        pltpu.VMEM((1, 8), i32),
        pltpu.VMEM((1, 8), i32),
        pltpu.VMEM((8, D), f32),
        pltpu.VMEM((CH, D), f32),
        pltpu.SemaphoreType.DMA((NBUF,)),
        pltpu.SemaphoreType.DMA((NBUF,)),
        pltpu.SemaphoreType.DMA((NBUF,)),
    ],
    name="sc_text_agg",
  )


def _sc_text(table, src, dst):
    return _sc_text_kernel()(table, src, dst)


BN = 1000


def _mlp_s0_body(q_ref, agg_ref, w_ref, b_ref, rel_ref, x1_ref, xr_ref):
    bb = pl.program_id(0)
    j = pl.program_id(1)
    w = w_ref[...]
    wb = w[D:]
    agg = agg_ref[0]
    y = jnp.dot(agg, wb, preferred_element_type=f32)
    csum = jnp.sum(w, axis=0)
    rid = j * BN + lax.broadcasted_iota(i32, (BN, 1), 0)
    y = y + jnp.where(rid == q_ref[bb], 1.0, 0.0) * csum[None, :]
    y = jnp.maximum(y + b_ref[...][None, :], 0.0)
    x1_ref[0] = y
    xr_ref[0] = rel_ref[...][:, None, :] * y[None]


def _mlp_s0(query, agg, w, bias, rel_next):
    return pl.pallas_call(
        _mlp_s0_body,
        grid=(B, N // BN),
        in_specs=[
            pl.BlockSpec(memory_space=pltpu.MemorySpace.SMEM),
            pl.BlockSpec((1, BN, D), lambda bb, j: (bb, j, 0)),
            pl.BlockSpec((2 * D, D), lambda bb, j: (0, 0)),
            pl.BlockSpec((D,), lambda bb, j: (0,)),
            pl.BlockSpec((TS, D), lambda bb, j: (0, 0)),
        ],
        out_specs=[
            pl.BlockSpec((1, BN, D), lambda bb, j: (bb, j, 0)),
            pl.BlockSpec((1, TS, BN, D), lambda bb, j: (bb, 0, j, 0)),
        ],
        out_shape=[
            jax.ShapeDtypeStruct((B, N, D), f32),
            jax.ShapeDtypeStruct((B, TS, N, D), f32),
        ],
    )(query, agg, w, bias, rel_next)


def _mlp_s1_body(q_ref, x_ref, agg_ref, w_ref, b_ref, h_ref):
    bb = pl.program_id(0)
    j = pl.program_id(1)
    w = w_ref[...]
    wt = w[:D]
    wb = w[D:]
    y = jnp.dot(x_ref[0], wt, preferred_element_type=f32)
    y = y + jnp.dot(agg_ref[0], wb, preferred_element_type=f32)
    csum = jnp.sum(wb, axis=0)
    rid = j * BN + lax.broadcasted_iota(i32, (BN, 1), 0)
    y = y + jnp.where(rid == q_ref[bb], 1.0, 0.0) * csum[None, :]
    h_ref[0] = jnp.maximum(y + b_ref[...][None, :], 0.0)


def _mlp_s1(query, x, agg, w, bias):
    return pl.pallas_call(
        _mlp_s1_body,
        grid=(B, N // BN),
        in_specs=[
            pl.BlockSpec(memory_space=pltpu.MemorySpace.SMEM),
            pl.BlockSpec((1, BN, D), lambda bb, j: (bb, j, 0)),
            pl.BlockSpec((1, BN, D), lambda bb, j: (bb, j, 0)),
            pl.BlockSpec((2 * D, D), lambda bb, j: (0, 0)),
            pl.BlockSpec((D,), lambda bb, j: (0,)),
        ],
        out_specs=pl.BlockSpec((1, BN, D), lambda bb, j: (bb, j, 0)),
        out_shape=jax.ShapeDtypeStruct((B, N, D), f32),
    )(query, x, agg, w, bias)


def _mlp_t_body(x_ref, part_ref, rti_ref, rel_ref, w_ref, b_ref, o_ref):
    w = w_ref[...]
    wt = w[:D]
    wb = w[D:]
    agg = (part_ref[0] + part_ref[1]) * rel_ref[...][0][None, :] \
        + rti_ref[...]
    y = jnp.dot(x_ref[...], wt, preferred_element_type=f32)
    y = y + jnp.dot(agg, wb, preferred_element_type=f32)
    o_ref[...] = jnp.maximum(y + b_ref[...][None, :], 0.0)


def _mlp_t(x, part, rti, rel, w, bias):
    return pl.pallas_call(
        _mlp_t_body,
        grid=(N // BN,),
        in_specs=[
            pl.BlockSpec((BN, D), lambda j: (j, 0)),
            pl.BlockSpec((NC, BN, D), lambda j: (0, j, 0)),
            pl.BlockSpec((BN, D), lambda j: (j, 0)),
            pl.BlockSpec((1, D), lambda j: (0, 0)),
            pl.BlockSpec((2 * D, D), lambda j: (0, 0)),
            pl.BlockSpec((D,), lambda j: (0,)),
        ],
        out_specs=pl.BlockSpec((BN, D), lambda j: (j, 0)),
        out_shape=jax.ShapeDtypeStruct((N, D), f32),
    )(x, part, rti, rel, w, bias)


def _fuse_body(h_ref, z_ref, w1_ref, b1_ref, w2_ref, b2_ref, o_ref):
    w1 = w1_ref[...]
    hid = jnp.dot(h_ref[0], w1[:D], preferred_element_type=f32)
    hid = hid + jnp.dot(z_ref[...], w1[D:], preferred_element_type=f32)
    hid = jnp.maximum(hid + b1_ref[...][None, :], 0.0)
    y = jnp.dot(hid, w2_ref[...], preferred_element_type=f32)
    o_ref[0] = y + b2_ref[...][None, :]


def _fuse(h, z, w1, b1, w2, b2):
    return pl.pallas_call(
        _fuse_body,
        grid=(B, N // BN),
        in_specs=[
            pl.BlockSpec((1, BN, D), lambda bb, j: (bb, j, 0)),
            pl.BlockSpec((BN, D), lambda bb, j: (j, 0)),
            pl.BlockSpec((2 * D, D), lambda bb, j: (0, 0)),
            pl.BlockSpec((D,), lambda bb, j: (0,)),
            pl.BlockSpec((D, D), lambda bb, j: (0, 0)),
            pl.BlockSpec((D,), lambda bb, j: (0,)),
        ],
        out_specs=pl.BlockSpec((1, BN, D), lambda bb, j: (bb, j, 0)),
        out_shape=jax.ShapeDtypeStruct((B, N, D), f32),
    )(h, z, w1, b1, w2, b2)


def kernel(query, edge_index, edge_type, text_edge_index, text_edge_type,
           rel_text_init, s_rel0, s_rel1, s_W0, s_b0, s_W1, s_b1,
           t_rel0, t_rel1, t_W0, t_b0, t_W1, t_b1,
           fuse_W1, fuse_b1, fuse_W2, fuse_b2):
    src_s = edge_index[0]
    dst_s = edge_index[1]
    src_t = text_edge_index[0]
    dst_t = text_edge_index[1]

    tab0 = jnp.zeros((B, TS, N, D), f32)
    tab0 = tab0.at[jnp.arange(B)[:, None], jnp.arange(TS)[None, :],
                   query[:, None], :].set(
        jnp.broadcast_to(s_rel0[None], (B, TS, D)))
    agg_s0 = _sc_struct(tab0.reshape(B * TS * N, D), src_s, dst_s, edge_type)
    x1, xrel1 = _mlp_s0(query, agg_s0.reshape(B, N, D), s_W0, s_b0, s_rel1)
    agg_s1 = _sc_struct(xrel1.reshape(B * TS * N, D), src_s, dst_s, edge_type)
    h = _mlp_s1(query, x1, agg_s1.reshape(B, N, D), s_W1, s_b1)

    part0 = _sc_text(rel_text_init, src_t, dst_t)
    x1t = _mlp_t(rel_text_init, part0.reshape(NC, N, D), rel_text_init,
                 t_rel0, t_W0, t_b0)
    part1 = _sc_text(x1t, src_t, dst_t)
    z = _mlp_t(x1t, part1.reshape(NC, N, D), rel_text_init,
               t_rel1, t_W1, t_b1)

    return _fuse(h, z, fuse_W1, fuse_b1, fuse_W2, fuse_b2)


# speedup vs baseline: 166.3054x; 1.0718x over previous
import functools

import jax
import jax.numpy as jnp
from jax import lax
from jax.experimental import pallas as pl
from jax.experimental.pallas import tpu as pltpu
from jax.experimental.pallas import tpu_sc as plsc

N = 10000
E = 160000
D = 128
B = 2
TS = 4
NC = 2
NSUB = 16
K = 96
NBUF_I = 4
NBUF_R = 3
WR_A = 632
WR_B = N - (NSUB - 1) * WR_A
CH = 40

f32 = jnp.float32
i32 = jnp.int32

@functools.cache
def _sc_mesh():
    return plsc.VectorSubcoreMesh(core_axis_name="c", subcore_axis_name="s",
                                  num_cores=NC, num_subcores=NSUB)


def _zero_vmem_rows(buf, nrows):
    z = jnp.zeros((16,), f32)

    @pl.loop(0, nrows)
    def _(r):
        for i in range(D // 16):
            buf[r, pl.ds(i * 16, 16)] = z


def _chunked(start, total, fn):
    nfull = total // CH
    tail = total - nfull * CH

    @pl.loop(0, nfull)
    def _(k):
        fn(start + k * CH, CH)

    if tail:
        fn(start + nfull * CH, tail)


def _per_tile_rows(s, fn):
    start = s * WR_A

    @pl.when(s < NSUB - 1)
    def _():
        fn(start, WR_A)

    @pl.when(s == NSUB - 1)
    def _():
        fn(start, WR_B)


def _zero_acc(acc, iobuf, s):
    _zero_vmem_rows(iobuf, CH)

    def z(off, sz):
        pltpu.sync_copy(iobuf.at[pl.ds(0, sz)], acc.at[pl.ds(off, sz)])

    _per_tile_rows(s, lambda start, total: _chunked(start, total, z))


def _acc_to_hbm(acc, iobuf, s, out, obase):
    def w(off, sz):
        pltpu.sync_copy(acc.at[pl.ds(off, sz)], iobuf.at[pl.ds(0, sz)])
        pltpu.sync_copy(iobuf.at[pl.ds(0, sz)],
                        out.at[pl.ds(obase + off, sz)])

    _per_tile_rows(s, lambda start, total: _chunked(start, total, w))


def _edge_pipeline(table, src, dst, et, acc, si, di, ti, gi, rows,
                   isem, gsem, ssem, base0, nfull, gidx):
    gref = gi if gidx is not None else si

    def islot_of(j):
        return lax.rem(j + 2 * NBUF_I, NBUF_I)

    def rslot_of(j):
        return lax.rem(j + 2 * NBUF_R, NBUF_R)

    def idx_issue(j):
        slot = islot_of(j)
        base = base0 + j * K
        pltpu.async_copy(src.at[pl.ds(base, K)], si.at[slot], isem.at[slot])
        pltpu.async_copy(dst.at[pl.ds(base, K)], di.at[slot], isem.at[slot])
        if et is not None:
            pltpu.async_copy(et.at[pl.ds(base, K)], ti.at[slot],
                             isem.at[slot])

    def idx_wait(j):
        slot = islot_of(j)
        base = base0 + j * K
        pltpu.make_async_copy(src.at[pl.ds(base, K)], si.at[slot],
                              isem.at[slot]).wait()
        pltpu.make_async_copy(dst.at[pl.ds(base, K)], di.at[slot],
                              isem.at[slot]).wait()
        if et is not None:
            pltpu.make_async_copy(et.at[pl.ds(base, K)], ti.at[slot],
                                  isem.at[slot]).wait()

    def gather_issue(j):
        pltpu.async_copy(table.at[gref.at[islot_of(j)]],
                         rows.at[rslot_of(j)], gsem.at[rslot_of(j)])

    def gather_wait(j):
        pltpu.make_async_copy(table.at[gref.at[islot_of(j)]],
                              rows.at[rslot_of(j)],
                              gsem.at[rslot_of(j)]).wait()

    def scatter_issue(j):
        pltpu.async_copy(rows.at[rslot_of(j)], acc.at[di.at[islot_of(j)]],
                         ssem.at[rslot_of(j)], add=True)

    def scatter_wait(j):
        pltpu.make_async_copy(rows.at[rslot_of(j)],
                              acc.at[di.at[islot_of(j)]],
                              ssem.at[rslot_of(j)]).wait()

    def stage_front(j):
        idx_wait(j)
        if gidx is not None:
            gidx(islot_of(j))
        gather_issue(j)

    idx_issue(0)
    idx_issue(1)
    stage_front(0)

    @pl.loop(0, nfull)
    def _(j):
        @pl.when(j >= 2)
        def _():
            scatter_wait(j - 2)

        @pl.when(j + 2 < nfull)
        def _():
            idx_issue(j + 2)

        @pl.when(j + 1 < nfull)
        def _():
            stage_front(j + 1)

        gather_wait(j)
        scatter_issue(j)

    scatter_wait(nfull - 2)
    scatter_wait(nfull - 1)


def _sc_struct_body(table, src, dst, et, out, acc, si, di, ti, gi, rows,
                    si_t, di_t, ti_t, gi_t, rows_t, iobuf, isem, gsem, ssem):
    c = lax.axis_index("c")
    s = lax.axis_index("s")
    b = c

    _zero_acc(acc, iobuf, s)
    plsc.subcore_barrier()

    ept = E // NSUB
    base0 = s * ept
    nfull = ept // K
    tail = ept - nfull * K

    def gidx(slot):
        for i in range(K // 16):
            sl = pl.ds(i * 16, 16)
            gi[slot, sl] = (b * TS + ti[slot, sl]) * N + si[slot, sl]

    _edge_pipeline(table, src, dst, et, acc, si, di, ti, gi, rows,
                   isem, gsem, ssem, base0, nfull, gidx)

    if tail:
        base = base0 + nfull * K
        pltpu.sync_copy(src.at[pl.ds(base, tail)], si_t.at[0])
        pltpu.sync_copy(dst.at[pl.ds(base, tail)], di_t.at[0])
        pltpu.sync_copy(et.at[pl.ds(base, tail)], ti_t.at[0])
        for i in range(tail // 16):
            sl = pl.ds(i * 16, 16)
            gi_t[0, sl] = (b * TS + ti_t[0, sl]) * N + si_t[0, sl]
        pltpu.async_copy(table.at[gi_t.at[0]], rows_t, gsem.at[0]).wait()
        pltpu.sync_copy(rows_t, acc.at[di_t.at[0]], add=True)

    plsc.subcore_barrier()
    _acc_to_hbm(acc, iobuf, s, out, b * N)


@functools.cache
def _sc_struct_kernel():
  return pl.kernel(
    _sc_struct_body,
    out_type=jax.ShapeDtypeStruct((B * N, D), f32),
    mesh=_sc_mesh(),
    scratch_types=[
        pltpu.VMEM_SHARED((N, D), f32),
        pltpu.VMEM((NBUF_I, K), i32),
        pltpu.VMEM((NBUF_I, K), i32),
        pltpu.VMEM((NBUF_I, K), i32),
        pltpu.VMEM((NBUF_I, K), i32),
        pltpu.VMEM((NBUF_R, K, D), f32),
        pltpu.VMEM((1, 16), i32),
        pltpu.VMEM((1, 16), i32),
        pltpu.VMEM((1, 16), i32),
        pltpu.VMEM((1, 16), i32),
        pltpu.VMEM((16, D), f32),
        pltpu.VMEM((CH, D), f32),
        pltpu.SemaphoreType.DMA((NBUF_I,)),
        pltpu.SemaphoreType.DMA((NBUF_R,)),
        pltpu.SemaphoreType.DMA((NBUF_R,)),
    ],
    name="sc_struct_agg",
  )


def _sc_struct(table, src, dst, et):
    return _sc_struct_kernel()(table, src, dst, et)


def _sc_text_body(table, src, dst, out, acc, si, di, rows,
                  si_t, di_t, rows_t, iobuf, isem, gsem, ssem):
    c = lax.axis_index("c")
    s = lax.axis_index("s")
    wid = s * NC + c

    _zero_acc(acc, iobuf, s)
    plsc.subcore_barrier()

    ept = E // (NC * NSUB)
    base0 = wid * ept
    nfull = ept // K
    tail = ept - nfull * K

    _edge_pipeline(table, src, dst, None, acc, si, di, None, None, rows,
                   isem, gsem, ssem, base0, nfull, None)

    if tail:
        base = base0 + nfull * K
        pltpu.sync_copy(src.at[pl.ds(base, tail)], si_t.at[0])
        pltpu.sync_copy(dst.at[pl.ds(base, tail)], di_t.at[0])
        pltpu.async_copy(table.at[si_t.at[0]], rows_t, gsem.at[0]).wait()
        pltpu.sync_copy(rows_t, acc.at[di_t.at[0]], add=True)

    plsc.subcore_barrier()
    _acc_to_hbm(acc, iobuf, s, out, c * N)


@functools.cache
def _sc_text_kernel():
  return pl.kernel(
    _sc_text_body,
    out_type=jax.ShapeDtypeStruct((NC * N, D), f32),
    mesh=_sc_mesh(),
    scratch_types=[
        pltpu.VMEM_SHARED((N, D), f32),
        pltpu.VMEM((NBUF_I, K), i32),
        pltpu.VMEM((NBUF_I, K), i32),
        pltpu.VMEM((NBUF_R, K, D), f32),
        pltpu.VMEM((1, 8), i32),
        pltpu.VMEM((1, 8), i32),
        pltpu.VMEM((8, D), f32),
        pltpu.VMEM((CH, D), f32),
        pltpu.SemaphoreType.DMA((NBUF_I,)),
        pltpu.SemaphoreType.DMA((NBUF_R,)),
        pltpu.SemaphoreType.DMA((NBUF_R,)),
    ],
    name="sc_text_agg",
  )


def _sc_text(table, src, dst):
    return _sc_text_kernel()(table, src, dst)


BN = 1000


def _mlp_s0_body(q_ref, agg_ref, w_ref, b_ref, rel_ref, x1_ref, xr_ref):
    bb = pl.program_id(0)
    j = pl.program_id(1)
    w = w_ref[...]
    wb = w[D:]
    agg = agg_ref[0]
    y = jnp.dot(agg, wb, preferred_element_type=f32)
    csum = jnp.sum(w, axis=0)
    rid = j * BN + lax.broadcasted_iota(i32, (BN, 1), 0)
    y = y + jnp.where(rid == q_ref[bb], 1.0, 0.0) * csum[None, :]
    y = jnp.maximum(y + b_ref[...][None, :], 0.0)
    x1_ref[0] = y
    xr_ref[0] = rel_ref[...][:, None, :] * y[None]


def _mlp_s0(query, agg, w, bias, rel_next):
    return pl.pallas_call(
        _mlp_s0_body,
        grid=(B, N // BN),
        in_specs=[
            pl.BlockSpec(memory_space=pltpu.MemorySpace.SMEM),
            pl.BlockSpec((1, BN, D), lambda bb, j: (bb, j, 0)),
            pl.BlockSpec((2 * D, D), lambda bb, j: (0, 0)),
            pl.BlockSpec((D,), lambda bb, j: (0,)),
            pl.BlockSpec((TS, D), lambda bb, j: (0, 0)),
        ],
        out_specs=[
            pl.BlockSpec((1, BN, D), lambda bb, j: (bb, j, 0)),
            pl.BlockSpec((1, TS, BN, D), lambda bb, j: (bb, 0, j, 0)),
        ],
        out_shape=[
            jax.ShapeDtypeStruct((B, N, D), f32),
            jax.ShapeDtypeStruct((B, TS, N, D), f32),
        ],
    )(query, agg, w, bias, rel_next)


def _mlp_s1_body(q_ref, x_ref, agg_ref, w_ref, b_ref, h_ref):
    bb = pl.program_id(0)
    j = pl.program_id(1)
    w = w_ref[...]
    wt = w[:D]
    wb = w[D:]
    y = jnp.dot(x_ref[0], wt, preferred_element_type=f32)
    y = y + jnp.dot(agg_ref[0], wb, preferred_element_type=f32)
    csum = jnp.sum(wb, axis=0)
    rid = j * BN + lax.broadcasted_iota(i32, (BN, 1), 0)
    y = y + jnp.where(rid == q_ref[bb], 1.0, 0.0) * csum[None, :]
    h_ref[0] = jnp.maximum(y + b_ref[...][None, :], 0.0)


def _mlp_s1(query, x, agg, w, bias):
    return pl.pallas_call(
        _mlp_s1_body,
        grid=(B, N // BN),
        in_specs=[
            pl.BlockSpec(memory_space=pltpu.MemorySpace.SMEM),
            pl.BlockSpec((1, BN, D), lambda bb, j: (bb, j, 0)),
            pl.BlockSpec((1, BN, D), lambda bb, j: (bb, j, 0)),
            pl.BlockSpec((2 * D, D), lambda bb, j: (0, 0)),
            pl.BlockSpec((D,), lambda bb, j: (0,)),
        ],
        out_specs=pl.BlockSpec((1, BN, D), lambda bb, j: (bb, j, 0)),
        out_shape=jax.ShapeDtypeStruct((B, N, D), f32),
    )(query, x, agg, w, bias)


def _mlp_t_body(x_ref, part_ref, rti_ref, rel_ref, w_ref, b_ref, o_ref):
    w = w_ref[...]
    wt = w[:D]
    wb = w[D:]
    agg = (part_ref[0] + part_ref[1]) * rel_ref[...][0][None, :] \
        + rti_ref[...]
    y = jnp.dot(x_ref[...], wt, preferred_element_type=f32)
    y = y + jnp.dot(agg, wb, preferred_element_type=f32)
    o_ref[...] = jnp.maximum(y + b_ref[...][None, :], 0.0)


def _mlp_t(x, part, rti, rel, w, bias):
    return pl.pallas_call(
        _mlp_t_body,
        grid=(N // BN,),
        in_specs=[
            pl.BlockSpec((BN, D), lambda j: (j, 0)),
            pl.BlockSpec((NC, BN, D), lambda j: (0, j, 0)),
            pl.BlockSpec((BN, D), lambda j: (j, 0)),
            pl.BlockSpec((1, D), lambda j: (0, 0)),
            pl.BlockSpec((2 * D, D), lambda j: (0, 0)),
            pl.BlockSpec((D,), lambda j: (0,)),
        ],
        out_specs=pl.BlockSpec((BN, D), lambda j: (j, 0)),
        out_shape=jax.ShapeDtypeStruct((N, D), f32),
    )(x, part, rti, rel, w, bias)


def _fuse_body(h_ref, z_ref, w1_ref, b1_ref, w2_ref, b2_ref, o_ref):
    w1 = w1_ref[...]
    hid = jnp.dot(h_ref[0], w1[:D], preferred_element_type=f32)
    hid = hid + jnp.dot(z_ref[...], w1[D:], preferred_element_type=f32)
    hid = jnp.maximum(hid + b1_ref[...][None, :], 0.0)
    y = jnp.dot(hid, w2_ref[...], preferred_element_type=f32)
    o_ref[0] = y + b2_ref[...][None, :]


def _fuse(h, z, w1, b1, w2, b2):
    return pl.pallas_call(
        _fuse_body,
        grid=(B, N // BN),
        in_specs=[
            pl.BlockSpec((1, BN, D), lambda bb, j: (bb, j, 0)),
            pl.BlockSpec((BN, D), lambda bb, j: (j, 0)),
            pl.BlockSpec((2 * D, D), lambda bb, j: (0, 0)),
            pl.BlockSpec((D,), lambda bb, j: (0,)),
            pl.BlockSpec((D, D), lambda bb, j: (0, 0)),
            pl.BlockSpec((D,), lambda bb, j: (0,)),
        ],
        out_specs=pl.BlockSpec((1, BN, D), lambda bb, j: (bb, j, 0)),
        out_shape=jax.ShapeDtypeStruct((B, N, D), f32),
    )(h, z, w1, b1, w2, b2)


def kernel(query, edge_index, edge_type, text_edge_index, text_edge_type,
           rel_text_init, s_rel0, s_rel1, s_W0, s_b0, s_W1, s_b1,
           t_rel0, t_rel1, t_W0, t_b0, t_W1, t_b1,
           fuse_W1, fuse_b1, fuse_W2, fuse_b2):
    src_s = edge_index[0]
    dst_s = edge_index[1]
    src_t = text_edge_index[0]
    dst_t = text_edge_index[1]

    tab0 = jnp.zeros((B, TS, N, D), f32)
    tab0 = tab0.at[jnp.arange(B)[:, None], jnp.arange(TS)[None, :],
                   query[:, None], :].set(
        jnp.broadcast_to(s_rel0[None], (B, TS, D)))
    agg_s0 = _sc_struct(tab0.reshape(B * TS * N, D), src_s, dst_s, edge_type)
    x1, xrel1 = _mlp_s0(query, agg_s0.reshape(B, N, D), s_W0, s_b0, s_rel1)
    agg_s1 = _sc_struct(xrel1.reshape(B * TS * N, D), src_s, dst_s, edge_type)
    h = _mlp_s1(query, x1, agg_s1.reshape(B, N, D), s_W1, s_b1)

    part0 = _sc_text(rel_text_init, src_t, dst_t)
    x1t = _mlp_t(rel_text_init, part0.reshape(NC, N, D), rel_text_init,
                 t_rel0, t_W0, t_b0)
    part1 = _sc_text(x1t, src_t, dst_t)
    z = _mlp_t(x1t, part1.reshape(NC, N, D), rel_text_init,
               t_rel1, t_W1, t_b1)

    return _fuse(h, z, fuse_W1, fuse_b1, fuse_W2, fuse_b2)


# speedup vs baseline: 207.3654x; 1.2469x over previous
import functools

import jax
import jax.numpy as jnp
from jax import lax
from jax.experimental import pallas as pl
from jax.experimental.pallas import tpu as pltpu
from jax.experimental.pallas import tpu_sc as plsc

N = 10000
E = 160000
D = 128
B = 2
TS = 4
NC = 2
NSUB = 16
K = 96
NBUF_I = 4
NBUF_R = 3
WR_A = 632
WR_B = N - (NSUB - 1) * WR_A
CH = 40

f32 = jnp.float32
i32 = jnp.int32

@functools.cache
def _sc_mesh():
    return plsc.VectorSubcoreMesh(core_axis_name="c", subcore_axis_name="s",
                                  num_cores=NC, num_subcores=NSUB)


def _zero_vmem_rows(buf, nrows):
    z = jnp.zeros((16,), f32)

    @pl.loop(0, nrows)
    def _(r):
        for i in range(D // 16):
            buf[r, pl.ds(i * 16, 16)] = z


def _chunked(start, total, fn):
    nfull = total // CH
    tail = total - nfull * CH

    @pl.loop(0, nfull)
    def _(k):
        fn(start + k * CH, CH)

    if tail:
        fn(start + nfull * CH, tail)


def _per_tile_rows(s, fn):
    start = s * WR_A

    @pl.when(s < NSUB - 1)
    def _():
        fn(start, WR_A)

    @pl.when(s == NSUB - 1)
    def _():
        fn(start, WR_B)


def _zero_acc(acc, iobuf, s):
    _zero_vmem_rows(iobuf, CH)

    def z(off, sz):
        pltpu.sync_copy(iobuf.at[pl.ds(0, sz)], acc.at[pl.ds(off, sz)])

    _per_tile_rows(s, lambda start, total: _chunked(start, total, z))


def _acc_to_hbm(acc, iobuf, s, out, obase):
    def w(off, sz):
        pltpu.sync_copy(acc.at[pl.ds(off, sz)], iobuf.at[pl.ds(0, sz)])
        pltpu.sync_copy(iobuf.at[pl.ds(0, sz)],
                        out.at[pl.ds(obase + off, sz)])

    _per_tile_rows(s, lambda start, total: _chunked(start, total, w))


def _edge_pipeline(table, src, dst, et, acc, si, di, ti, gi, rows,
                   isem, gsem, ssem, base0, nfull, gidx):
    gref = gi if gidx is not None else si

    def islot_of(j):
        return lax.rem(j + 2 * NBUF_I, NBUF_I)

    def rslot_of(j):
        return lax.rem(j + 2 * NBUF_R, NBUF_R)

    def idx_issue(j):
        slot = islot_of(j)
        base = base0 + j * K
        pltpu.async_copy(src.at[pl.ds(base, K)], si.at[slot], isem.at[slot])
        pltpu.async_copy(dst.at[pl.ds(base, K)], di.at[slot], isem.at[slot])
        if et is not None:
            pltpu.async_copy(et.at[pl.ds(base, K)], ti.at[slot],
                             isem.at[slot])

    def idx_wait(j):
        slot = islot_of(j)
        base = base0 + j * K
        pltpu.make_async_copy(src.at[pl.ds(base, K)], si.at[slot],
                              isem.at[slot]).wait()
        pltpu.make_async_copy(dst.at[pl.ds(base, K)], di.at[slot],
                              isem.at[slot]).wait()
        if et is not None:
            pltpu.make_async_copy(et.at[pl.ds(base, K)], ti.at[slot],
                                  isem.at[slot]).wait()

    def gather_issue(j):
        pltpu.async_copy(table.at[gref.at[islot_of(j)]],
                         rows.at[rslot_of(j)], gsem.at[rslot_of(j)])

    def gather_wait(j):
        pltpu.make_async_copy(table.at[gref.at[islot_of(j)]],
                              rows.at[rslot_of(j)],
                              gsem.at[rslot_of(j)]).wait()

    def scatter_issue(j):
        pltpu.async_copy(rows.at[rslot_of(j)], acc.at[di.at[islot_of(j)]],
                         ssem.at[rslot_of(j)], add=True)

    def scatter_wait(j):
        pltpu.make_async_copy(rows.at[rslot_of(j)],
                              acc.at[di.at[islot_of(j)]],
                              ssem.at[rslot_of(j)]).wait()

    def stage_front(j):
        idx_wait(j)
        if gidx is not None:
            gidx(islot_of(j))
        gather_issue(j)

    idx_issue(0)
    idx_issue(1)
    stage_front(0)

    @pl.loop(0, nfull)
    def _(j):
        @pl.when(j >= 2)
        def _():
            scatter_wait(j - 2)

        @pl.when(j + 2 < nfull)
        def _():
            idx_issue(j + 2)

        @pl.when(j + 1 < nfull)
        def _():
            stage_front(j + 1)

        gather_wait(j)
        scatter_issue(j)

    scatter_wait(nfull - 2)
    scatter_wait(nfull - 1)


def _sc_struct_body(table, src, dst, et, out, acc, si, di, ti, gi, rows,
                    si_t, di_t, ti_t, gi_t, rows_t, iobuf, isem, gsem, ssem):
    c = lax.axis_index("c")
    s = lax.axis_index("s")
    b = c

    _zero_acc(acc, iobuf, s)
    plsc.subcore_barrier()

    ept = E // NSUB
    base0 = s * ept
    nfull = ept // K
    tail = ept - nfull * K

    def gidx(slot):
        for i in range(K // 16):
            sl = pl.ds(i * 16, 16)
            gi[slot, sl] = (b * TS + ti[slot, sl]) * N + si[slot, sl]

    _edge_pipeline(table, src, dst, et, acc, si, di, ti, gi, rows,
                   isem, gsem, ssem, base0, nfull, gidx)

    if tail:
        base = base0 + nfull * K
        pltpu.sync_copy(src.at[pl.ds(base, tail)], si_t.at[0])
        pltpu.sync_copy(dst.at[pl.ds(base, tail)], di_t.at[0])
        pltpu.sync_copy(et.at[pl.ds(base, tail)], ti_t.at[0])
        for i in range(tail // 16):
            sl = pl.ds(i * 16, 16)
            gi_t[0, sl] = (b * TS + ti_t[0, sl]) * N + si_t[0, sl]
        pltpu.async_copy(table.at[gi_t.at[0]], rows_t, gsem.at[0]).wait()
        pltpu.sync_copy(rows_t, acc.at[di_t.at[0]], add=True)

    plsc.subcore_barrier()
    _acc_to_hbm(acc, iobuf, s, out, b * N)


@functools.cache
def _sc_struct_kernel():
  return pl.kernel(
    _sc_struct_body,
    out_type=jax.ShapeDtypeStruct((B * N, D), f32),
    mesh=_sc_mesh(),
    scratch_types=[
        pltpu.VMEM_SHARED((N, D), f32),
        pltpu.VMEM((NBUF_I, K), i32),
        pltpu.VMEM((NBUF_I, K), i32),
        pltpu.VMEM((NBUF_I, K), i32),
        pltpu.VMEM((NBUF_I, K), i32),
        pltpu.VMEM((NBUF_R, K, D), f32),
        pltpu.VMEM((1, 16), i32),
        pltpu.VMEM((1, 16), i32),
        pltpu.VMEM((1, 16), i32),
        pltpu.VMEM((1, 16), i32),
        pltpu.VMEM((16, D), f32),
        pltpu.VMEM((CH, D), f32),
        pltpu.SemaphoreType.DMA((NBUF_I,)),
        pltpu.SemaphoreType.DMA((NBUF_R,)),
        pltpu.SemaphoreType.DMA((NBUF_R,)),
    ],
    compiler_params=pltpu.CompilerParams(needs_layout_passes=False),
    name="sc_struct_agg",
  )


def _sc_struct(table, src, dst, et):
    return _sc_struct_kernel()(table, src, dst, et)


TSN = TS * N
CROW = 320


def _sc_count_body(qv, src, dst, et, out, acc2, qbuf, si, di, ti,
                   si_t, di_t, ti_t, isem):
    c = lax.axis_index("c")
    s = lax.axis_index("s")
    b = c

    z16 = jnp.zeros((16,), f32)

    @pl.loop(0, CROW)
    def _(r):
        for i in range(128 // 16):
            acc2[r, pl.ds(i * 16, 16)] = z16

    pltpu.sync_copy(qv.at[pl.ds(b * 16, 16)], qbuf.at[0])
    q = qbuf[0, pl.ds(0, 16)]

    ept = E // NSUB
    base0 = s * ept
    nfull = ept // K
    tail = ept - nfull * K

    def islot_of(j):
        return lax.rem(j + 2 * NBUF_I, NBUF_I)

    def idx_issue(j):
        slot = islot_of(j)
        base = base0 + j * K
        pltpu.async_copy(src.at[pl.ds(base, K)], si.at[slot], isem.at[slot])
        pltpu.async_copy(dst.at[pl.ds(base, K)], di.at[slot], isem.at[slot])
        pltpu.async_copy(et.at[pl.ds(base, K)], ti.at[slot], isem.at[slot])

    def idx_wait(j):
        slot = islot_of(j)
        base = base0 + j * K
        pltpu.make_async_copy(src.at[pl.ds(base, K)], si.at[slot],
                              isem.at[slot]).wait()
        pltpu.make_async_copy(dst.at[pl.ds(base, K)], di.at[slot],
                              isem.at[slot]).wait()
        pltpu.make_async_copy(et.at[pl.ds(base, K)], ti.at[slot],
                              isem.at[slot]).wait()

    def bump(sv, dv, tv):
        gi16 = tv * N + dv
        row = lax.shift_right_logical(gi16, 7)
        col = lax.bitwise_and(gi16, 127)
        v16 = jnp.where(sv == q, 1.0, 0.0)
        plsc.addupdate_scatter(acc2, [row, col], v16)

    def calc(j):
        slot = islot_of(j)
        for i in range(K // 16):
            sl = pl.ds(i * 16, 16)
            bump(si[slot, sl], di[slot, sl], ti[slot, sl])

    idx_issue(0)
    idx_issue(1)

    @pl.loop(0, nfull)
    def _(j):
        @pl.when(j + 2 < nfull)
        def _():
            idx_issue(j + 2)

        idx_wait(j)
        calc(j)

    if tail:
        base = base0 + nfull * K
        pltpu.sync_copy(src.at[pl.ds(base, tail)], si_t.at[0])
        pltpu.sync_copy(dst.at[pl.ds(base, tail)], di_t.at[0])
        pltpu.sync_copy(et.at[pl.ds(base, tail)], ti_t.at[0])
        for i in range(tail // 16):
            sl = pl.ds(i * 16, 16)
            bump(si_t[0, sl], di_t[0, sl], ti_t[0, sl])

    pltpu.sync_copy(acc2, out.at[pl.ds((b * NSUB + s) * CROW, CROW)])


@functools.cache
def _sc_count_kernel():
  return pl.kernel(
    _sc_count_body,
    out_type=jax.ShapeDtypeStruct((B * NSUB * CROW, 128), f32),
    mesh=_sc_mesh(),
    scratch_types=[
        pltpu.VMEM((CROW, 128), f32),
        pltpu.VMEM((1, 16), i32),
        pltpu.VMEM((NBUF_I, K), i32),
        pltpu.VMEM((NBUF_I, K), i32),
        pltpu.VMEM((NBUF_I, K), i32),
        pltpu.VMEM((1, 16), i32),
        pltpu.VMEM((1, 16), i32),
        pltpu.VMEM((1, 16), i32),
        pltpu.SemaphoreType.DMA((NBUF_I,)),
    ],
    compiler_params=pltpu.CompilerParams(needs_layout_passes=False),
    name="sc_struct_count",
  )


def _sc_count(qv, src, dst, et):
    return _sc_count_kernel()(qv, src, dst, et)


def _sc_text_body(table, src, dst, out, acc, si, di, rows,
                  si_t, di_t, rows_t, iobuf, isem, gsem, ssem):
    c = lax.axis_index("c")
    s = lax.axis_index("s")
    wid = s * NC + c

    _zero_acc(acc, iobuf, s)
    plsc.subcore_barrier()

    ept = E // (NC * NSUB)
    base0 = wid * ept
    nfull = ept // K
    tail = ept - nfull * K

    _edge_pipeline(table, src, dst, None, acc, si, di, None, None, rows,
                   isem, gsem, ssem, base0, nfull, None)

    if tail:
        base = base0 + nfull * K
        pltpu.sync_copy(src.at[pl.ds(base, tail)], si_t.at[0])
        pltpu.sync_copy(dst.at[pl.ds(base, tail)], di_t.at[0])
        pltpu.async_copy(table.at[si_t.at[0]], rows_t, gsem.at[0]).wait()
        pltpu.sync_copy(rows_t, acc.at[di_t.at[0]], add=True)

    plsc.subcore_barrier()
    _acc_to_hbm(acc, iobuf, s, out, c * N)


@functools.cache
def _sc_text_kernel():
  return pl.kernel(
    _sc_text_body,
    out_type=jax.ShapeDtypeStruct((NC * N, D), f32),
    mesh=_sc_mesh(),
    scratch_types=[
        pltpu.VMEM_SHARED((N, D), f32),
        pltpu.VMEM((NBUF_I, K), i32),
        pltpu.VMEM((NBUF_I, K), i32),
        pltpu.VMEM((NBUF_R, K, D), f32),
        pltpu.VMEM((1, 8), i32),
        pltpu.VMEM((1, 8), i32),
        pltpu.VMEM((8, D), f32),
        pltpu.VMEM((CH, D), f32),
        pltpu.SemaphoreType.DMA((NBUF_I,)),
        pltpu.SemaphoreType.DMA((NBUF_R,)),
        pltpu.SemaphoreType.DMA((NBUF_R,)),
    ],
    compiler_params=pltpu.CompilerParams(needs_layout_passes=False),
    name="sc_text_agg",
  )


def _sc_text(table, src, dst):
    return _sc_text_kernel()(table, src, dst)


BN = 1000


def _mlp_s0_body(q_ref, cnt_ref, w_ref, b_ref, rel0_ref, rel1_ref,
                 x1_ref, xr_ref):
    bb = pl.program_id(0)
    j = pl.program_id(1)
    w = w_ref[...]
    wb = w[D:]
    rw = jnp.dot(rel0_ref[...], wb, preferred_element_type=f32)
    cnt = jnp.sum(cnt_ref[0, 0], axis=0)
    y = lax.dot_general(cnt, rw, (((0,), (0,)), ((), ())),
                        preferred_element_type=f32)
    csum = jnp.sum(w, axis=0)
    rid = j * BN + lax.broadcasted_iota(i32, (BN, 1), 0)
    y = y + jnp.where(rid == q_ref[bb], 1.0, 0.0) * csum[None, :]
    y = jnp.maximum(y + b_ref[...][None, :], 0.0)
    x1_ref[0] = y
    xr_ref[0] = rel1_ref[...][:, None, :] * y[None]


def _mlp_s0(query, cnt, w, bias, rel0, rel_next):
    return pl.pallas_call(
        _mlp_s0_body,
        grid=(B, N // BN),
        in_specs=[
            pl.BlockSpec(memory_space=pltpu.MemorySpace.SMEM),
            pl.BlockSpec((1, 1, NSUB, TS, BN), lambda bb, j: (bb, j, 0, 0, 0)),
            pl.BlockSpec((2 * D, D), lambda bb, j: (0, 0)),
            pl.BlockSpec((D,), lambda bb, j: (0,)),
            pl.BlockSpec((TS, D), lambda bb, j: (0, 0)),
            pl.BlockSpec((TS, D), lambda bb, j: (0, 0)),
        ],
        out_specs=[
            pl.BlockSpec((1, BN, D), lambda bb, j: (bb, j, 0)),
            pl.BlockSpec((1, TS, BN, D), lambda bb, j: (bb, 0, j, 0)),
        ],
        out_shape=[
            jax.ShapeDtypeStruct((B, N, D), f32),
            jax.ShapeDtypeStruct((B, TS, N, D), f32),
        ],
    )(query, cnt, w, bias, rel0, rel_next)


def _mlp_s1_body(q_ref, x_ref, agg_ref, w_ref, b_ref, h_ref):
    bb = pl.program_id(0)
    j = pl.program_id(1)
    w = w_ref[...]
    wt = w[:D]
    wb = w[D:]
    y = jnp.dot(x_ref[0], wt, preferred_element_type=f32)
    y = y + jnp.dot(agg_ref[0], wb, preferred_element_type=f32)
    csum = jnp.sum(wb, axis=0)
    rid = j * BN + lax.broadcasted_iota(i32, (BN, 1), 0)
    y = y + jnp.where(rid == q_ref[bb], 1.0, 0.0) * csum[None, :]
    h_ref[0] = jnp.maximum(y + b_ref[...][None, :], 0.0)


def _mlp_s1(query, x, agg, w, bias):
    return pl.pallas_call(
        _mlp_s1_body,
        grid=(B, N // BN),
        in_specs=[
            pl.BlockSpec(memory_space=pltpu.MemorySpace.SMEM),
            pl.BlockSpec((1, BN, D), lambda bb, j: (bb, j, 0)),
            pl.BlockSpec((1, BN, D), lambda bb, j: (bb, j, 0)),
            pl.BlockSpec((2 * D, D), lambda bb, j: (0, 0)),
            pl.BlockSpec((D,), lambda bb, j: (0,)),
        ],
        out_specs=pl.BlockSpec((1, BN, D), lambda bb, j: (bb, j, 0)),
        out_shape=jax.ShapeDtypeStruct((B, N, D), f32),
    )(query, x, agg, w, bias)


def _mlp_t_body(x_ref, part_ref, rti_ref, rel_ref, w_ref, b_ref, o_ref):
    w = w_ref[...]
    wt = w[:D]
    wb = w[D:]
    agg = (part_ref[0] + part_ref[1]) * rel_ref[...][0][None, :] \
        + rti_ref[...]
    y = jnp.dot(x_ref[...], wt, preferred_element_type=f32)
    y = y + jnp.dot(agg, wb, preferred_element_type=f32)
    o_ref[...] = jnp.maximum(y + b_ref[...][None, :], 0.0)


def _mlp_t(x, part, rti, rel, w, bias):
    return pl.pallas_call(
        _mlp_t_body,
        grid=(N // BN,),
        in_specs=[
            pl.BlockSpec((BN, D), lambda j: (j, 0)),
            pl.BlockSpec((NC, BN, D), lambda j: (0, j, 0)),
            pl.BlockSpec((BN, D), lambda j: (j, 0)),
            pl.BlockSpec((1, D), lambda j: (0, 0)),
            pl.BlockSpec((2 * D, D), lambda j: (0, 0)),
            pl.BlockSpec((D,), lambda j: (0,)),
        ],
        out_specs=pl.BlockSpec((BN, D), lambda j: (j, 0)),
        out_shape=jax.ShapeDtypeStruct((N, D), f32),
    )(x, part, rti, rel, w, bias)


def _fuse_body(h_ref, z_ref, w1_ref, b1_ref, w2_ref, b2_ref, o_ref):
    w1 = w1_ref[...]
    hid = jnp.dot(h_ref[0], w1[:D], preferred_element_type=f32)
    hid = hid + jnp.dot(z_ref[...], w1[D:], preferred_element_type=f32)
    hid = jnp.maximum(hid + b1_ref[...][None, :], 0.0)
    y = jnp.dot(hid, w2_ref[...], preferred_element_type=f32)
    o_ref[0] = y + b2_ref[...][None, :]


def _fuse(h, z, w1, b1, w2, b2):
    return pl.pallas_call(
        _fuse_body,
        grid=(B, N // BN),
        in_specs=[
            pl.BlockSpec((1, BN, D), lambda bb, j: (bb, j, 0)),
            pl.BlockSpec((BN, D), lambda bb, j: (j, 0)),
            pl.BlockSpec((2 * D, D), lambda bb, j: (0, 0)),
            pl.BlockSpec((D,), lambda bb, j: (0,)),
            pl.BlockSpec((D, D), lambda bb, j: (0, 0)),
            pl.BlockSpec((D,), lambda bb, j: (0,)),
        ],
        out_specs=pl.BlockSpec((1, BN, D), lambda bb, j: (bb, j, 0)),
        out_shape=jax.ShapeDtypeStruct((B, N, D), f32),
    )(h, z, w1, b1, w2, b2)


def kernel(query, edge_index, edge_type, text_edge_index, text_edge_type,
           rel_text_init, s_rel0, s_rel1, s_W0, s_b0, s_W1, s_b1,
           t_rel0, t_rel1, t_W0, t_b0, t_W1, t_b1,
           fuse_W1, fuse_b1, fuse_W2, fuse_b2):
    src_s = edge_index[0]
    dst_s = edge_index[1]
    src_t = text_edge_index[0]
    dst_t = text_edge_index[1]

    qv = jnp.broadcast_to(query[:, None], (B, 16)).reshape(B * 16)
    cnt = _sc_count(qv, src_s, dst_s, edge_type)
    cnt4 = (cnt.reshape(B, NSUB, CROW * 128)[:, :, :TSN]
            .reshape(B, NSUB, TS, N // BN, BN).transpose(0, 3, 1, 2, 4))
    x1, xrel1 = _mlp_s0(query, cnt4, s_W0, s_b0, s_rel0, s_rel1)
    agg_s1 = _sc_struct(xrel1.reshape(B * TS * N, D), src_s, dst_s, edge_type)
    h = _mlp_s1(query, x1, agg_s1.reshape(B, N, D), s_W1, s_b1)

    part0 = _sc_text(rel_text_init, src_t, dst_t)
    x1t = _mlp_t(rel_text_init, part0.reshape(NC, N, D), rel_text_init,
                 t_rel0, t_W0, t_b0)
    part1 = _sc_text(x1t, src_t, dst_t)
    z = _mlp_t(x1t, part1.reshape(NC, N, D), rel_text_init,
               t_rel1, t_W1, t_b1)

    return _fuse(h, z, fuse_W1, fuse_b1, fuse_W2, fuse_b2)


# speedup vs baseline: 211.7692x; 1.0212x over previous
import functools

import jax
import jax.numpy as jnp
from jax import lax
from jax.experimental import pallas as pl
from jax.experimental.pallas import tpu as pltpu
from jax.experimental.pallas import tpu_sc as plsc

N = 10000
E = 160000
D = 128
B = 2
TS = 4
NC = 2
NSUB = 16
K = 96
NBUF_I = 4
NBUF_R = 3
WR_A = 632
WR_B = N - (NSUB - 1) * WR_A
CH = 40
BN = 1000

f32 = jnp.float32
i32 = jnp.int32

@functools.cache
def _sc_mesh():
    return plsc.VectorSubcoreMesh(core_axis_name="c", subcore_axis_name="s",
                                  num_cores=NC, num_subcores=NSUB)


def _zero_vmem_rows(buf, nrows):
    z = jnp.zeros((16,), f32)

    @pl.loop(0, nrows)
    def _(r):
        for i in range(D // 16):
            buf[r, pl.ds(i * 16, 16)] = z


def _chunked(start, total, fn):
    nfull = total // CH
    tail = total - nfull * CH

    @pl.loop(0, nfull)
    def _(k):
        fn(start + k * CH, CH)

    if tail:
        fn(start + nfull * CH, tail)


def _per_tile_rows(s, fn):
    start = s * WR_A

    @pl.when(s < NSUB - 1)
    def _():
        fn(start, WR_A)

    @pl.when(s == NSUB - 1)
    def _():
        fn(start, WR_B)


def _zero_acc(acc, iobuf, s):
    _zero_vmem_rows(iobuf, CH)

    def z(off, sz):
        pltpu.sync_copy(iobuf.at[pl.ds(0, sz)], acc.at[pl.ds(off, sz)])

    _per_tile_rows(s, lambda start, total: _chunked(start, total, z))


def _acc_to_hbm(acc, iobuf, s, out, obase):
    def w(off, sz):
        pltpu.sync_copy(acc.at[pl.ds(off, sz)], iobuf.at[pl.ds(0, sz)])
        pltpu.sync_copy(iobuf.at[pl.ds(0, sz)],
                        out.at[pl.ds(obase + off, sz)])

    _per_tile_rows(s, lambda start, total: _chunked(start, total, w))


def _edge_pipeline(table, src, dst, et, acc, si, di, ti, gi, rows,
                   isem, gsem, ssem, base0, nfull, gidx):
    gref = gi if gidx is not None else si

    def islot_of(j):
        return lax.rem(j + 2 * NBUF_I, NBUF_I)

    def rslot_of(j):
        return lax.rem(j + 2 * NBUF_R, NBUF_R)

    def idx_issue(j):
        slot = islot_of(j)
        base = base0 + j * K
        pltpu.async_copy(src.at[pl.ds(base, K)], si.at[slot], isem.at[slot])
        pltpu.async_copy(dst.at[pl.ds(base, K)], di.at[slot], isem.at[slot])
        if et is not None:
            pltpu.async_copy(et.at[pl.ds(base, K)], ti.at[slot],
                             isem.at[slot])

    def idx_wait(j):
        slot = islot_of(j)
        base = base0 + j * K
        pltpu.make_async_copy(src.at[pl.ds(base, K)], si.at[slot],
                              isem.at[slot]).wait()
        pltpu.make_async_copy(dst.at[pl.ds(base, K)], di.at[slot],
                              isem.at[slot]).wait()
        if et is not None:
            pltpu.make_async_copy(et.at[pl.ds(base, K)], ti.at[slot],
                                  isem.at[slot]).wait()

    def gather_issue(j):
        pltpu.async_copy(table.at[gref.at[islot_of(j)]],
                         rows.at[rslot_of(j)], gsem.at[rslot_of(j)])

    def gather_wait(j):
        pltpu.make_async_copy(table.at[gref.at[islot_of(j)]],
                              rows.at[rslot_of(j)],
                              gsem.at[rslot_of(j)]).wait()

    def scatter_issue(j):
        pltpu.async_copy(rows.at[rslot_of(j)], acc.at[di.at[islot_of(j)]],
                         ssem.at[rslot_of(j)], add=True)

    def scatter_wait(j):
        pltpu.make_async_copy(rows.at[rslot_of(j)],
                              acc.at[di.at[islot_of(j)]],
                              ssem.at[rslot_of(j)]).wait()

    def stage_front(j):
        idx_wait(j)
        if gidx is not None:
            gidx(islot_of(j))
        gather_issue(j)

    idx_issue(0)
    idx_issue(1)
    stage_front(0)

    @pl.loop(0, nfull)
    def _(j):
        @pl.when(j >= 2)
        def _():
            scatter_wait(j - 2)

        @pl.when(j + 2 < nfull)
        def _():
            idx_issue(j + 2)

        @pl.when(j + 1 < nfull)
        def _():
            stage_front(j + 1)

        gather_wait(j)
        scatter_issue(j)

    scatter_wait(nfull - 2)
    scatter_wait(nfull - 1)


def _sc_struct_body(table, src, dst, et, out, acc, si, di, ti, gi, rows,
                    si_t, di_t, ti_t, gi_t, rows_t, iobuf, isem, gsem, ssem):
    c = lax.axis_index("c")
    s = lax.axis_index("s")
    b = c

    _zero_acc(acc, iobuf, s)
    plsc.subcore_barrier()

    ept = E // NSUB
    base0 = s * ept
    nfull = ept // K
    tail = ept - nfull * K

    def gidx(slot):
        for i in range(K // 16):
            sl = pl.ds(i * 16, 16)
            gi[slot, sl] = (b * TS + ti[slot, sl]) * N + si[slot, sl]

    _edge_pipeline(table, src, dst, et, acc, si, di, ti, gi, rows,
                   isem, gsem, ssem, base0, nfull, gidx)

    if tail:
        base = base0 + nfull * K
        pltpu.sync_copy(src.at[pl.ds(base, tail)], si_t.at[0])
        pltpu.sync_copy(dst.at[pl.ds(base, tail)], di_t.at[0])
        pltpu.sync_copy(et.at[pl.ds(base, tail)], ti_t.at[0])
        for i in range(tail // 16):
            sl = pl.ds(i * 16, 16)
            gi_t[0, sl] = (b * TS + ti_t[0, sl]) * N + si_t[0, sl]
        pltpu.async_copy(table.at[gi_t.at[0]], rows_t, gsem.at[0]).wait()
        pltpu.sync_copy(rows_t, acc.at[di_t.at[0]], add=True)

    plsc.subcore_barrier()
    _acc_to_hbm(acc, iobuf, s, out, b * N)


@functools.cache
def _sc_struct_kernel():
  return pl.kernel(
    _sc_struct_body,
    out_type=jax.ShapeDtypeStruct((B * N, D), f32),
    mesh=_sc_mesh(),
    scratch_types=[
        pltpu.VMEM_SHARED((N, D), f32),
        pltpu.VMEM((NBUF_I, K), i32),
        pltpu.VMEM((NBUF_I, K), i32),
        pltpu.VMEM((NBUF_I, K), i32),
        pltpu.VMEM((NBUF_I, K), i32),
        pltpu.VMEM((NBUF_R, K, D), f32),
        pltpu.VMEM((1, 16), i32),
        pltpu.VMEM((1, 16), i32),
        pltpu.VMEM((1, 16), i32),
        pltpu.VMEM((1, 16), i32),
        pltpu.VMEM((16, D), f32),
        pltpu.VMEM((CH, D), f32),
        pltpu.SemaphoreType.DMA((NBUF_I,)),
        pltpu.SemaphoreType.DMA((NBUF_R,)),
        pltpu.SemaphoreType.DMA((NBUF_R,)),
    ],
    compiler_params=pltpu.CompilerParams(needs_layout_passes=False),
    name="sc_struct_agg",
  )


def _sc_struct(table, src, dst, et):
    return _sc_struct_kernel()(table, src, dst, et)


TSN = TS * N
CROW = 320


def _sc_count_body(qv, src, dst, et, out, acc2, qbuf, si, di, ti,
                   si_t, di_t, ti_t, isem):
    c = lax.axis_index("c")
    s = lax.axis_index("s")
    b = c

    z16 = jnp.zeros((16,), f32)

    @pl.loop(0, CROW)
    def _(r):
        for i in range(128 // 16):
            acc2[r, pl.ds(i * 16, 16)] = z16

    pltpu.sync_copy(qv.at[pl.ds(b * 16, 16)], qbuf.at[0])
    q = qbuf[0, pl.ds(0, 16)]

    ept = E // NSUB
    base0 = s * ept
    nfull = ept // K
    tail = ept - nfull * K

    def islot_of(j):
        return lax.rem(j + 2 * NBUF_I, NBUF_I)

    def idx_issue(j):
        slot = islot_of(j)
        base = base0 + j * K
        pltpu.async_copy(src.at[pl.ds(base, K)], si.at[slot], isem.at[slot])
        pltpu.async_copy(dst.at[pl.ds(base, K)], di.at[slot], isem.at[slot])
        pltpu.async_copy(et.at[pl.ds(base, K)], ti.at[slot], isem.at[slot])

    def idx_wait(j):
        slot = islot_of(j)
        base = base0 + j * K
        pltpu.make_async_copy(src.at[pl.ds(base, K)], si.at[slot],
                              isem.at[slot]).wait()
        pltpu.make_async_copy(dst.at[pl.ds(base, K)], di.at[slot],
                              isem.at[slot]).wait()
        pltpu.make_async_copy(et.at[pl.ds(base, K)], ti.at[slot],
                              isem.at[slot]).wait()

    def bump(sv, dv, tv):
        nb = lax.shift_right_logical(dv * 8389, 23)
        gi16 = (nb * TS + tv) * BN + (dv - nb * BN)
        row = lax.shift_right_logical(gi16, 7)
        col = lax.bitwise_and(gi16, 127)
        v16 = jnp.where(sv == q, 1.0, 0.0)
        plsc.addupdate_scatter(acc2, [row, col], v16)

    def calc(j):
        slot = islot_of(j)
        for i in range(K // 16):
            sl = pl.ds(i * 16, 16)
            bump(si[slot, sl], di[slot, sl], ti[slot, sl])

    idx_issue(0)
    idx_issue(1)

    @pl.loop(0, nfull)
    def _(j):
        @pl.when(j + 2 < nfull)
        def _():
            idx_issue(j + 2)

        idx_wait(j)
        calc(j)

    if tail:
        base = base0 + nfull * K
        pltpu.sync_copy(src.at[pl.ds(base, tail)], si_t.at[0])
        pltpu.sync_copy(dst.at[pl.ds(base, tail)], di_t.at[0])
        pltpu.sync_copy(et.at[pl.ds(base, tail)], ti_t.at[0])
        for i in range(tail // 16):
            sl = pl.ds(i * 16, 16)
            bump(si_t[0, sl], di_t[0, sl], ti_t[0, sl])

    pltpu.sync_copy(acc2, out.at[pl.ds((b * NSUB + s) * CROW, CROW)])


@functools.cache
def _sc_count_kernel():
  return pl.kernel(
    _sc_count_body,
    out_type=jax.ShapeDtypeStruct((B * NSUB * CROW, 128), f32),
    mesh=_sc_mesh(),
    scratch_types=[
        pltpu.VMEM((CROW, 128), f32),
        pltpu.VMEM((1, 16), i32),
        pltpu.VMEM((NBUF_I, K), i32),
        pltpu.VMEM((NBUF_I, K), i32),
        pltpu.VMEM((NBUF_I, K), i32),
        pltpu.VMEM((1, 16), i32),
        pltpu.VMEM((1, 16), i32),
        pltpu.VMEM((1, 16), i32),
        pltpu.SemaphoreType.DMA((NBUF_I,)),
    ],
    compiler_params=pltpu.CompilerParams(needs_layout_passes=False),
    name="sc_struct_count",
  )


def _sc_count(qv, src, dst, et):
    return _sc_count_kernel()(qv, src, dst, et)


def _sc_text_body(table, src, dst, out, acc, si, di, rows,
                  si_t, di_t, rows_t, iobuf, isem, gsem, ssem):
    c = lax.axis_index("c")
    s = lax.axis_index("s")
    wid = s * NC + c

    _zero_acc(acc, iobuf, s)
    plsc.subcore_barrier()

    ept = E // (NC * NSUB)
    base0 = wid * ept
    nfull = ept // K
    tail = ept - nfull * K

    _edge_pipeline(table, src, dst, None, acc, si, di, None, None, rows,
                   isem, gsem, ssem, base0, nfull, None)

    if tail:
        base = base0 + nfull * K
        pltpu.sync_copy(src.at[pl.ds(base, tail)], si_t.at[0])
        pltpu.sync_copy(dst.at[pl.ds(base, tail)], di_t.at[0])
        pltpu.async_copy(table.at[si_t.at[0]], rows_t, gsem.at[0]).wait()
        pltpu.sync_copy(rows_t, acc.at[di_t.at[0]], add=True)

    plsc.subcore_barrier()
    _acc_to_hbm(acc, iobuf, s, out, c * N)


@functools.cache
def _sc_text_kernel():
  return pl.kernel(
    _sc_text_body,
    out_type=jax.ShapeDtypeStruct((NC * N, D), f32),
    mesh=_sc_mesh(),
    scratch_types=[
        pltpu.VMEM_SHARED((N, D), f32),
        pltpu.VMEM((NBUF_I, K), i32),
        pltpu.VMEM((NBUF_I, K), i32),
        pltpu.VMEM((NBUF_R, K, D), f32),
        pltpu.VMEM((1, 8), i32),
        pltpu.VMEM((1, 8), i32),
        pltpu.VMEM((8, D), f32),
        pltpu.VMEM((CH, D), f32),
        pltpu.SemaphoreType.DMA((NBUF_I,)),
        pltpu.SemaphoreType.DMA((NBUF_R,)),
        pltpu.SemaphoreType.DMA((NBUF_R,)),
    ],
    compiler_params=pltpu.CompilerParams(needs_layout_passes=False),
    name="sc_text_agg",
  )


def _sc_text(table, src, dst):
    return _sc_text_kernel()(table, src, dst)


def _mlp_a_body(q_ref, cnt_ref, sw_ref, sb_ref, r0_ref, r1_ref,
                part_ref, rti_ref, tr_ref, tw_ref, tb_ref,
                x1_ref, xr_ref, x1t_ref):
    j = pl.program_id(0)
    sw = sw_ref[...]
    swb = sw[D:]
    rw = jnp.dot(r0_ref[...], swb, preferred_element_type=f32)
    csum = jnp.sum(sw, axis=0)
    rid = j * BN + lax.broadcasted_iota(i32, (BN, 1), 0)
    rel1 = r1_ref[...]
    for bb in range(B):
        cnt = jnp.sum(cnt_ref[bb, :, 0], axis=0)
        y = lax.dot_general(cnt, rw, (((0,), (0,)), ((), ())),
                            preferred_element_type=f32)
        y = y + jnp.where(rid == q_ref[bb], 1.0, 0.0) * csum[None, :]
        y = jnp.maximum(y + sb_ref[...][None, :], 0.0)
        x1_ref[bb] = y
        xr_ref[bb] = rel1[:, None, :] * y[None]
    tw = tw_ref[...]
    rti = rti_ref[...]
    zagg = (part_ref[0] + part_ref[1]) * tr_ref[...][0][None, :] + rti
    yz = jnp.dot(rti, tw[:D], preferred_element_type=f32)
    yz = yz + jnp.dot(zagg, tw[D:], preferred_element_type=f32)
    x1t_ref[...] = jnp.maximum(yz + tb_ref[...][None, :], 0.0)


def _mlp_a(query, cnt, sw, sb, rel0, rel1, part, rti, trel, tw, tb):
    return pl.pallas_call(
        _mlp_a_body,
        grid=(N // BN,),
        in_specs=[
            pl.BlockSpec(memory_space=pltpu.MemorySpace.SMEM),
            pl.BlockSpec((B, NSUB, 1, TS, BN), lambda j: (0, 0, j, 0, 0)),
            pl.BlockSpec((2 * D, D), lambda j: (0, 0)),
            pl.BlockSpec((D,), lambda j: (0,)),
            pl.BlockSpec((TS, D), lambda j: (0, 0)),
            pl.BlockSpec((TS, D), lambda j: (0, 0)),
            pl.BlockSpec((NC, BN, D), lambda j: (0, j, 0)),
            pl.BlockSpec((BN, D), lambda j: (j, 0)),
            pl.BlockSpec((1, D), lambda j: (0, 0)),
            pl.BlockSpec((2 * D, D), lambda j: (0, 0)),
            pl.BlockSpec((D,), lambda j: (0,)),
        ],
        out_specs=[
            pl.BlockSpec((B, BN, D), lambda j: (0, j, 0)),
            pl.BlockSpec((B, TS, BN, D), lambda j: (0, 0, j, 0)),
            pl.BlockSpec((BN, D), lambda j: (j, 0)),
        ],
        out_shape=[
            jax.ShapeDtypeStruct((B, N, D), f32),
            jax.ShapeDtypeStruct((B, TS, N, D), f32),
            jax.ShapeDtypeStruct((N, D), f32),
        ],
    )(query, cnt, sw, sb, rel0, rel1, part, rti, trel, tw, tb)


def _mlp_b_body(q_ref, x1_ref, agg_ref, sw_ref, sb_ref,
                x1t_ref, part_ref, rti_ref, tr_ref, tw_ref, tb_ref,
                fw1_ref, fb1_ref, fw2_ref, fb2_ref, o_ref):
    j = pl.program_id(0)
    tw = tw_ref[...]
    zagg = (part_ref[0] + part_ref[1]) * tr_ref[...][0][None, :] \
        + rti_ref[...]
    z = jnp.dot(x1t_ref[...], tw[:D], preferred_element_type=f32)
    z = z + jnp.dot(zagg, tw[D:], preferred_element_type=f32)
    z = jnp.maximum(z + tb_ref[...][None, :], 0.0)
    fw1 = fw1_ref[...]
    zf = jnp.dot(z, fw1[D:], preferred_element_type=f32)
    sw = sw_ref[...]
    swt = sw[:D]
    swb = sw[D:]
    csumb = jnp.sum(swb, axis=0)
    rid = j * BN + lax.broadcasted_iota(i32, (BN, 1), 0)
    for bb in range(B):
        h = jnp.dot(x1_ref[bb], swt, preferred_element_type=f32)
        h = h + jnp.dot(agg_ref[bb], swb, preferred_element_type=f32)
        h = h + jnp.where(rid == q_ref[bb], 1.0, 0.0) * csumb[None, :]
        h = jnp.maximum(h + sb_ref[...][None, :], 0.0)
        hid = jnp.dot(h, fw1[:D], preferred_element_type=f32) + zf
        hid = jnp.maximum(hid + fb1_ref[...][None, :], 0.0)
        y = jnp.dot(hid, fw2_ref[...], preferred_element_type=f32)
        o_ref[bb] = y + fb2_ref[...][None, :]


def _mlp_b(query, x1, agg, sw, sb, x1t, part, rti, trel, tw, tb,
           fw1, fb1, fw2, fb2):
    return pl.pallas_call(
        _mlp_b_body,
        grid=(N // BN,),
        in_specs=[
            pl.BlockSpec(memory_space=pltpu.MemorySpace.SMEM),
            pl.BlockSpec((B, BN, D), lambda j: (0, j, 0)),
            pl.BlockSpec((B, BN, D), lambda j: (0, j, 0)),
            pl.BlockSpec((2 * D, D), lambda j: (0, 0)),
            pl.BlockSpec((D,), lambda j: (0,)),
            pl.BlockSpec((BN, D), lambda j: (j, 0)),
            pl.BlockSpec((NC, BN, D), lambda j: (0, j, 0)),
            pl.BlockSpec((BN, D), lambda j: (j, 0)),
            pl.BlockSpec((1, D), lambda j: (0, 0)),
            pl.BlockSpec((2 * D, D), lambda j: (0, 0)),
            pl.BlockSpec((D,), lambda j: (0,)),
            pl.BlockSpec((2 * D, D), lambda j: (0, 0)),
            pl.BlockSpec((D,), lambda j: (0,)),
            pl.BlockSpec((D, D), lambda j: (0, 0)),
            pl.BlockSpec((D,), lambda j: (0,)),
        ],
        out_specs=pl.BlockSpec((B, BN, D), lambda j: (0, j, 0)),
        out_shape=jax.ShapeDtypeStruct((B, N, D), f32),
    )(query, x1, agg, sw, sb, x1t, part, rti, trel, tw, tb,
      fw1, fb1, fw2, fb2)


def kernel(query, edge_index, edge_type, text_edge_index, text_edge_type,
           rel_text_init, s_rel0, s_rel1, s_W0, s_b0, s_W1, s_b1,
           t_rel0, t_rel1, t_W0, t_b0, t_W1, t_b1,
           fuse_W1, fuse_b1, fuse_W2, fuse_b2):
    src_s = edge_index[0]
    dst_s = edge_index[1]
    src_t = text_edge_index[0]
    dst_t = text_edge_index[1]

    qv = jnp.broadcast_to(query[:, None], (B, 16)).reshape(B * 16)
    cnt = _sc_count(qv, src_s, dst_s, edge_type)
    cnt5 = (cnt.reshape(B, NSUB, CROW * 128)[:, :, :TSN]
            .reshape(B, NSUB, N // BN, TS, BN))
    part0 = _sc_text(rel_text_init, src_t, dst_t)
    x1, xrel1, x1t = _mlp_a(query, cnt5, s_W0, s_b0, s_rel0, s_rel1,
                            part0.reshape(NC, N, D), rel_text_init,
                            t_rel0, t_W0, t_b0)
    agg_s1 = _sc_struct(xrel1.reshape(B * TS * N, D), src_s, dst_s, edge_type)
    part1 = _sc_text(x1t, src_t, dst_t)
    return _mlp_b(query, x1, agg_s1.reshape(B, N, D), s_W1, s_b1,
                  x1t, part1.reshape(NC, N, D), rel_text_init,
                  t_rel1, t_W1, t_b1, fuse_W1, fuse_b1, fuse_W2, fuse_b2)
